# Initial kernel scaffold; baseline (speedup 1.0000x reference)
import jax
import jax.numpy as jnp
from jax.experimental import pallas as pl


def kernel(x, pos, edge_index, batch, params):
    raise NotImplementedError("write your pallas kernel here")


# speedup vs baseline: 1.5825x; 1.5825x over previous
import functools

import jax
import jax.numpy as jnp
from jax import lax
from jax.experimental import pallas as pl
from jax.experimental.pallas import tpu as pltpu
from jax.experimental.pallas import tpu_sc as plsc

N = 10000
E = 320000
D_IN = 128
H = 64
OUT = 128
NPAD = 10240

NC = 2
NS = 16
NW = NC * NS
EPW = E // NW
K = 80
NIT = EPW // K
RPT = NPAD // NS

_mesh = plsc.VectorSubcoreMesh(core_axis_name="c", subcore_axis_name="s")
_sc_params = pltpu.CompilerParams(use_tc_tiling_on_sc=False)

f32 = jnp.float32
i32 = jnp.int32


@functools.partial(
    pl.kernel,
    out_type=(
        jax.ShapeDtypeStruct((E, H), f32),
        jax.ShapeDtypeStruct((E, H), f32),
        jax.ShapeDtypeStruct((E, 8), f32),
        jax.ShapeDtypeStruct((E, 8), f32),
    ),
    mesh=_mesh,
    compiler_params=_sc_params,
    scratch_types=[
        pltpu.VMEM((K,), i32),
        pltpu.VMEM((K,), i32),
        pltpu.VMEM((K, H), f32),
        pltpu.VMEM((K, H), f32),
        pltpu.VMEM((K, 8), f32),
        pltpu.VMEM((K, 8), f32),
        pltpu.SemaphoreType.DMA,
    ],
)
def _sc_gather4(h_hbm, c_hbm, row_hbm, col_hbm,
                xr_hbm, xc_hbm, cr_hbm, cc_hbm,
                ir, ic, bxr, bxc, bcr, bcc, sem):
    wid = lax.axis_index("s") * NC + lax.axis_index("c")
    base = wid * EPW

    def body(j, carry):
        off = base + j * K
        pltpu.sync_copy(row_hbm.at[pl.ds(off, K)], ir)
        pltpu.sync_copy(col_hbm.at[pl.ds(off, K)], ic)
        c1 = pltpu.async_copy(h_hbm.at[ir], bxr, sem)
        c2 = pltpu.async_copy(h_hbm.at[ic], bxc, sem)
        c3 = pltpu.async_copy(c_hbm.at[ir], bcr, sem)
        c4 = pltpu.async_copy(c_hbm.at[ic], bcc, sem)
        c1.wait(); c2.wait(); c3.wait(); c4.wait()
        pltpu.sync_copy(bxr, xr_hbm.at[pl.ds(off, K)])
        pltpu.sync_copy(bxc, xc_hbm.at[pl.ds(off, K)])
        pltpu.sync_copy(bcr, cr_hbm.at[pl.ds(off, K)])
        pltpu.sync_copy(bcc, cc_hbm.at[pl.ds(off, K)])
        return carry

    lax.fori_loop(0, NIT, body, 0)


@functools.partial(
    pl.kernel,
    out_type=(
        jax.ShapeDtypeStruct((E, 8), f32),
        jax.ShapeDtypeStruct((E, 8), f32),
    ),
    mesh=_mesh,
    compiler_params=_sc_params,
    scratch_types=[
        pltpu.VMEM((K,), i32),
        pltpu.VMEM((K,), i32),
        pltpu.VMEM((K, 8), f32),
        pltpu.VMEM((K, 8), f32),
        pltpu.SemaphoreType.DMA,
    ],
)
def _sc_gather2(c_hbm, row_hbm, col_hbm, cr_hbm, cc_hbm,
                ir, ic, bcr, bcc, sem):
    wid = lax.axis_index("s") * NC + lax.axis_index("c")
    base = wid * EPW

    def body(j, carry):
        off = base + j * K
        pltpu.sync_copy(row_hbm.at[pl.ds(off, K)], ir)
        pltpu.sync_copy(col_hbm.at[pl.ds(off, K)], ic)
        c1 = pltpu.async_copy(c_hbm.at[ir], bcr, sem)
        c2 = pltpu.async_copy(c_hbm.at[ic], bcc, sem)
        c1.wait(); c2.wait()
        pltpu.sync_copy(bcr, cr_hbm.at[pl.ds(off, K)])
        pltpu.sync_copy(bcc, cc_hbm.at[pl.ds(off, K)])
        return carry

    lax.fori_loop(0, NIT, body, 0)


def _make_sc_scatter(W):
    @functools.partial(
        pl.kernel,
        out_type=jax.ShapeDtypeStruct((2, NPAD, W), f32),
        mesh=_mesh,
        compiler_params=_sc_params,
        scratch_types=[
            pltpu.VMEM((K,), i32),
            pltpu.VMEM((K, W), f32),
            pltpu.VMEM_SHARED((NPAD, W), f32),
            pltpu.SemaphoreType.DMA,
        ],
    )
    def scat(val_hbm, row_hbm, zero_hbm, out_hbm, idx, vbuf, acc, sem):
        cid = lax.axis_index("c")
        sid = lax.axis_index("s")
        wid = sid * NC + cid
        pltpu.sync_copy(zero_hbm.at[pl.ds(sid * RPT, RPT)],
                        acc.at[pl.ds(sid * RPT, RPT)])
        plsc.subcore_barrier()
        base = wid * EPW

        def body(j, carry):
            off = base + j * K
            pltpu.sync_copy(row_hbm.at[pl.ds(off, K)], idx)
            pltpu.sync_copy(val_hbm.at[pl.ds(off, K)], vbuf)
            pltpu.async_copy(vbuf, acc.at[idx], sem, add=True).wait()
            return carry

        lax.fori_loop(0, NIT, body, 0)
        plsc.subcore_barrier()
        pltpu.sync_copy(acc.at[pl.ds(sid * RPT, RPT)],
                        out_hbm.at[cid, pl.ds(sid * RPT, RPT)])

    return scat


_sc_scatter8 = _make_sc_scatter(8)
_sc_scatter64 = _make_sc_scatter(H)


EB = 512
NBK = 1024


def _full(shape):
    return pl.BlockSpec(shape, lambda *_: tuple(0 for _ in shape))


def _rows(bs, w):
    return pl.BlockSpec((bs, w), lambda i, *_: (i, 0))


def _proj_body(x_ref, wt, b, o_ref):
    o_ref[...] = (jnp.dot(x_ref[...], wt[...], preferred_element_type=f32)
                  + b[...])


def _edge_a_body(xr, xc, cr, cc, w1r, w1c, w1p, b1, w2t, eww, msc, dout):
    rel = cr[...] - cc[...]
    u = (jnp.dot(xr[...], w1r[...], preferred_element_type=f32)
         + jnp.dot(xc[...], w1c[...], preferred_element_type=f32)
         + jnp.dot(rel, w1p[...], preferred_element_type=f32) + b1[...])
    u = u * jax.nn.sigmoid(u)
    d = jnp.dot(u, w2t[...], preferred_element_type=f32)
    nrm = jnp.maximum(jnp.sqrt(jnp.sum(d * d, axis=-1, keepdims=True)), 1e-8)
    ew = jax.nn.sigmoid(jnp.sum(rel * eww[...], axis=-1, keepdims=True)
                        + msc[0, 1])
    dout[...] = d * ((msc[0, 0] * ew) / nrm)


def _edge_b_body(xr, xc, cr, cc, e1r, e1c, e1p, b1, e2t, b2, eout):
    rel = cr[...] - cc[...]
    u = (jnp.dot(xr[...], e1r[...], preferred_element_type=f32)
         + jnp.dot(xc[...], e1c[...], preferred_element_type=f32)
         + jnp.dot(rel, e1p[...], preferred_element_type=f32) + b1[...])
    u = u * jax.nn.sigmoid(u)
    v = jnp.dot(u, e2t[...], preferred_element_type=f32) + b2[...]
    eout[...] = v * jax.nn.sigmoid(v)


def _cupd_body(c, q0, q1, cout):
    cout[...] = c[...] + q0[...] + q1[...]


def _node_body(h, p0, p1, n1h, n1a, b1, n2t, b2, hout):
    agg = p0[...] + p1[...]
    u = (jnp.dot(h[...], n1h[...], preferred_element_type=f32)
         + jnp.dot(agg, n1a[...], preferred_element_type=f32) + b1[...])
    u = u * jax.nn.sigmoid(u)
    hout[...] = h[...] + jnp.dot(u, n2t[...], preferred_element_type=f32) + b2[...]


def _head_body(h, bcol, g1t, g1b, lng, lnb, g2t, g2b, g3t, g3b, wot, bo,
               out, m_ref, d_ref, p_ref):
    ph = pl.program_id(0)
    i = pl.program_id(1)
    G = pl.num_programs(1)

    @pl.when(jnp.logical_and(ph == 0, i == 0))
    def _():
        m_ref[...] = jnp.full((1, 8), -1e30, f32)

    @pl.when(jnp.logical_and(ph == 1, i == 0))
    def _():
        d_ref[...] = jnp.zeros((8, 8), f32)
        p_ref[...] = jnp.zeros((8, H), f32)

    g = jnp.dot(h[...], g1t[...], preferred_element_type=f32) + g1b[...]
    mu = jnp.mean(g, axis=-1, keepdims=True)
    var = jnp.mean((g - mu) ** 2, axis=-1, keepdims=True)
    g = (g - mu) * lax.rsqrt(var + 1e-5) * lng[...] + lnb[...]
    g = jnp.maximum(g, 0.0)
    q = jnp.maximum(jnp.dot(g, g2t[...], preferred_element_type=f32)
                    + g2b[...], 0.0)
    s = jnp.dot(q, g3t[...], preferred_element_type=f32) + g3b[...]
    s0 = s[:, 0:1]
    oh = bcol[...] == lax.broadcasted_iota(i32, (1, 8), 1)

    @pl.when(ph == 0)
    def _():
        sm = jnp.where(oh, s0, -1e30)
        bm = jnp.max(sm, axis=0, keepdims=True)
        m_ref[...] = jnp.maximum(m_ref[...], bm)

    @pl.when(ph == 1)
    def _():
        w = jnp.where(oh, jnp.exp(s0 - m_ref[...]), 0.0)
        dn = (((0,), (0,)), ((), ()))
        d_ref[...] += lax.dot_general(w, jnp.ones((NBK, 8), f32), dn,
                                      preferred_element_type=f32)
        p_ref[...] += lax.dot_general(w, h[...], dn,
                                      preferred_element_type=f32)

    @pl.when(jnp.logical_and(ph == 1, i == G - 1))
    def _():
        pooled = p_ref[...] / d_ref[:, 0:1]
        out[...] = (jnp.dot(pooled, wot[...], preferred_element_type=f32)
                    + bo[...])


def _proj_call(xp, wt, b):
    return pl.pallas_call(
        _proj_body,
        grid=(NPAD // NBK,),
        in_specs=[_rows(NBK, D_IN), _full((D_IN, H)), _full((1, H))],
        out_specs=_rows(NBK, H),
        out_shape=jax.ShapeDtypeStruct((NPAD, H), f32),
    )(xp, wt, b)


def _edge_a_call(xr, xc, cr, cc, w1r, w1c, w1p, b1, w2t, eww, msc):
    return pl.pallas_call(
        _edge_a_body,
        grid=(E // EB,),
        in_specs=[_rows(EB, H), _rows(EB, H), _rows(EB, 8), _rows(EB, 8),
                  _full((H, 2 * H)), _full((H, 2 * H)), _full((8, 2 * H)),
                  _full((1, 2 * H)), _full((2 * H, 8)), _full((1, 8)),
                  pl.BlockSpec(memory_space=pltpu.SMEM)],
        out_specs=_rows(EB, 8),
        out_shape=jax.ShapeDtypeStruct((E, 8), f32),
    )(xr, xc, cr, cc, w1r, w1c, w1p, b1, w2t, eww, msc)


def _edge_b_call(xr, xc, cr, cc, e1r, e1c, e1p, b1, e2t, b2):
    return pl.pallas_call(
        _edge_b_body,
        grid=(E // EB,),
        in_specs=[_rows(EB, H), _rows(EB, H), _rows(EB, 8), _rows(EB, 8),
                  _full((H, 2 * H)), _full((H, 2 * H)), _full((8, 2 * H)),
                  _full((1, 2 * H)), _full((2 * H, H)), _full((1, H))],
        out_specs=_rows(EB, H),
        out_shape=jax.ShapeDtypeStruct((E, H), f32),
    )(xr, xc, cr, cc, e1r, e1c, e1p, b1, e2t, b2)


def _cupd_call(c, q0, q1):
    return pl.pallas_call(
        _cupd_body,
        grid=(NPAD // NBK,),
        in_specs=[_rows(NBK, 8)] * 3,
        out_specs=_rows(NBK, 8),
        out_shape=jax.ShapeDtypeStruct((NPAD, 8), f32),
    )(c, q0, q1)


def _node_call(h, p0, p1, n1h, n1a, b1, n2t, b2):
    return pl.pallas_call(
        _node_body,
        grid=(NPAD // NBK,),
        in_specs=[_rows(NBK, H), _rows(NBK, H), _rows(NBK, H),
                  _full((H, 2 * H)), _full((H, 2 * H)), _full((1, 2 * H)),
                  _full((2 * H, H)), _full((1, H))],
        out_specs=_rows(NBK, H),
        out_shape=jax.ShapeDtypeStruct((NPAD, H), f32),
    )(h, p0, p1, n1h, n1a, b1, n2t, b2)


def _head_call(h, bcol, g1t, g1b, lng, lnb, g2t, g2b, g3t, g3b, wot, bo):
    return pl.pallas_call(
        _head_body,
        grid=(2, NPAD // NBK),
        in_specs=[pl.BlockSpec((NBK, H), lambda p, i: (i, 0)),
                  pl.BlockSpec((NBK, 1), lambda p, i: (i, 0)),
                  _full((H, H)), _full((1, H)), _full((1, H)), _full((1, H)),
                  _full((H, H // 2)), _full((1, H // 2)),
                  _full((H // 2, 8)), _full((1, 8)),
                  _full((H, OUT)), _full((1, OUT))],
        out_specs=pl.BlockSpec((8, OUT), lambda p, i: (0, 0)),
        out_shape=jax.ShapeDtypeStruct((8, OUT), f32),
        scratch_shapes=[pltpu.VMEM((1, 8), f32), pltpu.VMEM((8, 8), f32),
                        pltpu.VMEM((8, H), f32)],
    )(h, bcol, g1t, g1b, lng, lnb, g2t, g2b, g3t, g3b, wot, bo)


def _r(v, n):
    return v.reshape(1, n)


def kernel(x, pos, edge_index, batch, params):
    row = edge_index[0].astype(i32)
    col = edge_index[1].astype(i32)

    xp = jnp.pad(x, ((0, NPAD - N), (0, 0)))
    coords = jnp.pad(pos, ((0, NPAD - N), (0, 5)))
    bcol = jnp.pad(batch.astype(i32), (0, NPAD - N),
                   constant_values=127).reshape(NPAD, 1)
    zeros8 = jnp.zeros((NPAD, 8), f32)
    zeros64 = jnp.zeros((NPAD, H), f32)

    wi, bi = params['input_proj']
    h = _proj_call(xp, wi.T, _r(bi, H))

    for p in params['layers']:
        w1, b1 = p['coord1']
        w2, b2 = p['coord2']
        we, be = p['ew']
        we1, be1 = p['edge1']
        we2, be2 = p['edge2']
        wn1, bn1 = p['node1']
        wn2, bn2 = p['node2']

        w1p = jnp.pad(w1[:, 2 * H:].T, ((0, 5), (0, 0)))
        w2t = jnp.pad(w2.T, ((0, 0), (0, 5)))
        eww = jnp.pad(we, ((0, 0), (0, 5)))
        msc = jnp.stack([p['scale'][0], be[0]]).reshape(1, 2)
        msc = jnp.pad(msc, ((0, 0), (0, 6)))
        we1p = jnp.pad(we1[:, 2 * H:].T, ((0, 5), (0, 0)))

        xr, xc, cr, cc = _sc_gather4(h, coords, row, col)
        delta = _edge_a_call(xr, xc, cr, cc,
                             w1[:, :H].T, w1[:, H:2 * H].T, w1p,
                             _r(b1, 2 * H), w2t, eww, msc)
        q = _sc_scatter8(delta, row, zeros8)
        coords = _cupd_call(coords, q[0], q[1])
        cr2, cc2 = _sc_gather2(coords, row, col)
        e = _edge_b_call(xr, xc, cr2, cc2,
                         we1[:, :H].T, we1[:, H:2 * H].T, we1p,
                         _r(be1, 2 * H), we2.T, _r(be2, H))
        ag = _sc_scatter64(e, row, zeros64)
        h = _node_call(h, ag[0], ag[1],
                       wn1[:, :H].T, wn1[:, H:].T, _r(bn1, 2 * H),
                       wn2.T, _r(bn2, H))

    wg1, bg1 = params['gate1']
    wg2, bg2 = params['gate2']
    wg3, bg3 = params['gate3']
    wo, bo = params['output_proj']
    g3t = jnp.pad(wg3.T, ((0, 0), (0, 7)))
    g3b = jnp.pad(bg3.reshape(1, 1), ((0, 0), (0, 7)))
    return _head_call(h, bcol, wg1.T, _r(bg1, H),
                      _r(params['ln_g'], H), _r(params['ln_b'], H),
                      wg2.T, _r(bg2, H // 2), g3t, g3b,
                      wo.T, _r(bo, OUT))


# speedup vs baseline: 1.6266x; 1.0278x over previous
import functools

import jax
import jax.numpy as jnp
from jax import lax
from jax.experimental import pallas as pl
from jax.experimental.pallas import tpu as pltpu
from jax.experimental.pallas import tpu_sc as plsc

N = 10000
E = 320000
D_IN = 128
H = 64
OUT = 128
NPAD = 10240

NC = 2
NS = 16
NW = NC * NS
EPW = E // NW
K = 80
NIT = EPW // K
RPT = NPAD // NS

_mesh = plsc.VectorSubcoreMesh(core_axis_name="c", subcore_axis_name="s")
_sc_params = pltpu.CompilerParams(use_tc_tiling_on_sc=False)

f32 = jnp.float32
i32 = jnp.int32


@functools.partial(
    pl.kernel,
    out_type=(
        jax.ShapeDtypeStruct((E, 2 * H), f32),
        jax.ShapeDtypeStruct((E, 8), f32),
        jax.ShapeDtypeStruct((E, 8), f32),
    ),
    mesh=_mesh,
    compiler_params=_sc_params,
    scratch_types=[
        pltpu.VMEM((K,), i32),
        pltpu.VMEM((K,), i32),
        pltpu.VMEM((K, H), f32),
        pltpu.VMEM((K, H), f32),
        pltpu.VMEM((K, 2 * H), f32),
        pltpu.VMEM((K, 8), f32),
        pltpu.VMEM((K, 8), f32),
        pltpu.SemaphoreType.DMA,
    ],
)
def _sc_gather4(h_hbm, c_hbm, row_hbm, col_hbm,
                xg_hbm, cr_hbm, cc_hbm,
                ir, ic, bxr, bxc, bxg, bcr, bcc, sem):
    wid = lax.axis_index("s") * NC + lax.axis_index("c")
    base = wid * EPW
    L = 16

    def body(j, carry):
        off = base + j * K
        pltpu.sync_copy(row_hbm.at[pl.ds(off, K)], ir)
        pltpu.sync_copy(col_hbm.at[pl.ds(off, K)], ic)
        c1 = pltpu.async_copy(h_hbm.at[ir], bxr, sem)
        c2 = pltpu.async_copy(h_hbm.at[ic], bxc, sem)
        c3 = pltpu.async_copy(c_hbm.at[ir], bcr, sem)
        c4 = pltpu.async_copy(c_hbm.at[ic], bcc, sem)
        c1.wait(); c2.wait(); c3.wait(); c4.wait()

        def ilv(r, cr2_):
            for q in range(H // L):
                bxg[r, pl.ds(L * q, L)] = bxr[r, pl.ds(L * q, L)]
                bxg[r, pl.ds(H + L * q, L)] = bxc[r, pl.ds(L * q, L)]
            return cr2_

        lax.fori_loop(0, K, ilv, 0)
        pltpu.sync_copy(bxg, xg_hbm.at[pl.ds(off, K)])
        pltpu.sync_copy(bcr, cr_hbm.at[pl.ds(off, K)])
        pltpu.sync_copy(bcc, cc_hbm.at[pl.ds(off, K)])
        return carry

    lax.fori_loop(0, NIT, body, 0)


@functools.partial(
    pl.kernel,
    out_type=(
        jax.ShapeDtypeStruct((E, 8), f32),
        jax.ShapeDtypeStruct((E, 8), f32),
    ),
    mesh=_mesh,
    compiler_params=_sc_params,
    scratch_types=[
        pltpu.VMEM((K,), i32),
        pltpu.VMEM((K,), i32),
        pltpu.VMEM((K, 8), f32),
        pltpu.VMEM((K, 8), f32),
        pltpu.SemaphoreType.DMA,
    ],
)
def _sc_gather2(c_hbm, row_hbm, col_hbm, cr_hbm, cc_hbm,
                ir, ic, bcr, bcc, sem):
    wid = lax.axis_index("s") * NC + lax.axis_index("c")
    base = wid * EPW

    def body(j, carry):
        off = base + j * K
        pltpu.sync_copy(row_hbm.at[pl.ds(off, K)], ir)
        pltpu.sync_copy(col_hbm.at[pl.ds(off, K)], ic)
        c1 = pltpu.async_copy(c_hbm.at[ir], bcr, sem)
        c2 = pltpu.async_copy(c_hbm.at[ic], bcc, sem)
        c1.wait(); c2.wait()
        pltpu.sync_copy(bcr, cr_hbm.at[pl.ds(off, K)])
        pltpu.sync_copy(bcc, cc_hbm.at[pl.ds(off, K)])
        return carry

    lax.fori_loop(0, NIT, body, 0)


def _make_sc_scatter(W):
    @functools.partial(
        pl.kernel,
        out_type=jax.ShapeDtypeStruct((2, NPAD, W), f32),
        mesh=_mesh,
        compiler_params=_sc_params,
        scratch_types=[
            pltpu.VMEM((K,), i32),
            pltpu.VMEM((K, W), f32),
            pltpu.VMEM_SHARED((NPAD, W), f32),
            pltpu.SemaphoreType.DMA,
        ],
    )
    def scat(val_hbm, row_hbm, zero_hbm, out_hbm, idx, vbuf, acc, sem):
        cid = lax.axis_index("c")
        sid = lax.axis_index("s")
        wid = sid * NC + cid
        pltpu.sync_copy(zero_hbm.at[pl.ds(sid * RPT, RPT)],
                        acc.at[pl.ds(sid * RPT, RPT)])
        plsc.subcore_barrier()
        base = wid * EPW

        def body(j, carry):
            off = base + j * K
            pltpu.sync_copy(row_hbm.at[pl.ds(off, K)], idx)
            pltpu.sync_copy(val_hbm.at[pl.ds(off, K)], vbuf)
            pltpu.async_copy(vbuf, acc.at[idx], sem, add=True).wait()
            return carry

        lax.fori_loop(0, NIT, body, 0)
        plsc.subcore_barrier()
        pltpu.sync_copy(acc.at[pl.ds(sid * RPT, RPT)],
                        out_hbm.at[cid, pl.ds(sid * RPT, RPT)])

    return scat


_sc_scatter8 = _make_sc_scatter(8)
_sc_scatter64 = _make_sc_scatter(H)


EB = 512
NBK = 1024


def _full(shape):
    return pl.BlockSpec(shape, lambda *_: tuple(0 for _ in shape))


def _rows(bs, w):
    return pl.BlockSpec((bs, w), lambda i, *_: (i, 0))


def _proj_body(x_ref, wt, b, o_ref):
    o_ref[...] = (jnp.dot(x_ref[...], wt[...], preferred_element_type=f32)
                  + b[...])


def _edge_a_body(xg, cr, cc, w1x, w1p, b1, w2t, eww, msc, dout):
    rel = cr[...] - cc[...]
    u = (jnp.dot(xg[...], w1x[...], preferred_element_type=f32)
         + jnp.dot(rel, w1p[...], preferred_element_type=f32) + b1[...])
    u = u * jax.nn.sigmoid(u)
    d = jnp.dot(u, w2t[...], preferred_element_type=f32)
    nrm = jnp.maximum(jnp.sqrt(jnp.sum(d * d, axis=-1, keepdims=True)), 1e-8)
    ew = jax.nn.sigmoid(jnp.sum(rel * eww[...], axis=-1, keepdims=True)
                        + msc[0, 1])
    dout[...] = d * ((msc[0, 0] * ew) / nrm)


def _edge_b_body(xg, cr, cc, e1x, e1p, b1, e2t, b2, eout):
    rel = cr[...] - cc[...]
    u = (jnp.dot(xg[...], e1x[...], preferred_element_type=f32)
         + jnp.dot(rel, e1p[...], preferred_element_type=f32) + b1[...])
    u = u * jax.nn.sigmoid(u)
    v = jnp.dot(u, e2t[...], preferred_element_type=f32) + b2[...]
    eout[...] = v * jax.nn.sigmoid(v)


def _cupd_body(c, q0, q1, cout):
    cout[...] = c[...] + q0[...] + q1[...]


def _node_body(h, p0, p1, n1h, n1a, b1, n2t, b2, hout):
    agg = p0[...] + p1[...]
    u = (jnp.dot(h[...], n1h[...], preferred_element_type=f32)
         + jnp.dot(agg, n1a[...], preferred_element_type=f32) + b1[...])
    u = u * jax.nn.sigmoid(u)
    hout[...] = h[...] + jnp.dot(u, n2t[...], preferred_element_type=f32) + b2[...]


def _head_body(h, bcol, g1t, g1b, lng, lnb, g2t, g2b, g3t, g3b, wot, bo,
               out, m_ref, d_ref, p_ref):
    ph = pl.program_id(0)
    i = pl.program_id(1)
    G = pl.num_programs(1)

    @pl.when(jnp.logical_and(ph == 0, i == 0))
    def _():
        m_ref[...] = jnp.full((1, 8), -1e30, f32)

    @pl.when(jnp.logical_and(ph == 1, i == 0))
    def _():
        d_ref[...] = jnp.zeros((8, 8), f32)
        p_ref[...] = jnp.zeros((8, H), f32)

    g = jnp.dot(h[...], g1t[...], preferred_element_type=f32) + g1b[...]
    mu = jnp.mean(g, axis=-1, keepdims=True)
    var = jnp.mean((g - mu) ** 2, axis=-1, keepdims=True)
    g = (g - mu) * lax.rsqrt(var + 1e-5) * lng[...] + lnb[...]
    g = jnp.maximum(g, 0.0)
    q = jnp.maximum(jnp.dot(g, g2t[...], preferred_element_type=f32)
                    + g2b[...], 0.0)
    s = jnp.dot(q, g3t[...], preferred_element_type=f32) + g3b[...]
    s0 = s[:, 0:1]
    oh = bcol[...] == lax.broadcasted_iota(i32, (1, 8), 1)

    @pl.when(ph == 0)
    def _():
        sm = jnp.where(oh, s0, -1e30)
        bm = jnp.max(sm, axis=0, keepdims=True)
        m_ref[...] = jnp.maximum(m_ref[...], bm)

    @pl.when(ph == 1)
    def _():
        w = jnp.where(oh, jnp.exp(s0 - m_ref[...]), 0.0)
        dn = (((0,), (0,)), ((), ()))
        d_ref[...] += lax.dot_general(w, jnp.ones((NBK, 8), f32), dn,
                                      preferred_element_type=f32)
        p_ref[...] += lax.dot_general(w, h[...], dn,
                                      preferred_element_type=f32)

    @pl.when(jnp.logical_and(ph == 1, i == G - 1))
    def _():
        pooled = p_ref[...] / d_ref[:, 0:1]
        out[...] = (jnp.dot(pooled, wot[...], preferred_element_type=f32)
                    + bo[...])


def _proj_call(xp, wt, b):
    return pl.pallas_call(
        _proj_body,
        grid=(NPAD // NBK,),
        in_specs=[_rows(NBK, D_IN), _full((D_IN, H)), _full((1, H))],
        out_specs=_rows(NBK, H),
        out_shape=jax.ShapeDtypeStruct((NPAD, H), f32),
    )(xp, wt, b)


def _edge_a_call(xg, cr, cc, w1x, w1p, b1, w2t, eww, msc):
    return pl.pallas_call(
        _edge_a_body,
        grid=(E // EB,),
        in_specs=[_rows(EB, 2 * H), _rows(EB, 8), _rows(EB, 8),
                  _full((2 * H, 2 * H)), _full((8, 2 * H)),
                  _full((1, 2 * H)), _full((2 * H, 8)), _full((1, 8)),
                  pl.BlockSpec(memory_space=pltpu.SMEM)],
        out_specs=_rows(EB, 8),
        out_shape=jax.ShapeDtypeStruct((E, 8), f32),
    )(xg, cr, cc, w1x, w1p, b1, w2t, eww, msc)


def _edge_b_call(xg, cr, cc, e1x, e1p, b1, e2t, b2):
    return pl.pallas_call(
        _edge_b_body,
        grid=(E // EB,),
        in_specs=[_rows(EB, 2 * H), _rows(EB, 8), _rows(EB, 8),
                  _full((2 * H, 2 * H)), _full((8, 2 * H)),
                  _full((1, 2 * H)), _full((2 * H, H)), _full((1, H))],
        out_specs=_rows(EB, H),
        out_shape=jax.ShapeDtypeStruct((E, H), f32),
    )(xg, cr, cc, e1x, e1p, b1, e2t, b2)


def _cupd_call(c, q0, q1):
    return pl.pallas_call(
        _cupd_body,
        grid=(NPAD // NBK,),
        in_specs=[_rows(NBK, 8)] * 3,
        out_specs=_rows(NBK, 8),
        out_shape=jax.ShapeDtypeStruct((NPAD, 8), f32),
    )(c, q0, q1)


def _node_call(h, p0, p1, n1h, n1a, b1, n2t, b2):
    return pl.pallas_call(
        _node_body,
        grid=(NPAD // NBK,),
        in_specs=[_rows(NBK, H), _rows(NBK, H), _rows(NBK, H),
                  _full((H, 2 * H)), _full((H, 2 * H)), _full((1, 2 * H)),
                  _full((2 * H, H)), _full((1, H))],
        out_specs=_rows(NBK, H),
        out_shape=jax.ShapeDtypeStruct((NPAD, H), f32),
    )(h, p0, p1, n1h, n1a, b1, n2t, b2)


def _head_call(h, bcol, g1t, g1b, lng, lnb, g2t, g2b, g3t, g3b, wot, bo):
    return pl.pallas_call(
        _head_body,
        grid=(2, NPAD // NBK),
        in_specs=[pl.BlockSpec((NBK, H), lambda p, i: (i, 0)),
                  pl.BlockSpec((NBK, 1), lambda p, i: (i, 0)),
                  _full((H, H)), _full((1, H)), _full((1, H)), _full((1, H)),
                  _full((H, H // 2)), _full((1, H // 2)),
                  _full((H // 2, 8)), _full((1, 8)),
                  _full((H, OUT)), _full((1, OUT))],
        out_specs=pl.BlockSpec((8, OUT), lambda p, i: (0, 0)),
        out_shape=jax.ShapeDtypeStruct((8, OUT), f32),
        scratch_shapes=[pltpu.VMEM((1, 8), f32), pltpu.VMEM((8, 8), f32),
                        pltpu.VMEM((8, H), f32)],
    )(h, bcol, g1t, g1b, lng, lnb, g2t, g2b, g3t, g3b, wot, bo)


def _r(v, n):
    return v.reshape(1, n)


def kernel(x, pos, edge_index, batch, params):
    row = edge_index[0].astype(i32)
    col = edge_index[1].astype(i32)

    xp = jnp.pad(x, ((0, NPAD - N), (0, 0)))
    coords = jnp.pad(pos, ((0, NPAD - N), (0, 5)))
    bcol = jnp.pad(batch.astype(i32), (0, NPAD - N),
                   constant_values=127).reshape(NPAD, 1)
    zeros8 = jnp.zeros((NPAD, 8), f32)
    zeros64 = jnp.zeros((NPAD, H), f32)

    wi, bi = params['input_proj']
    h = _proj_call(xp, wi.T, _r(bi, H))

    for p in params['layers']:
        w1, b1 = p['coord1']
        w2, b2 = p['coord2']
        we, be = p['ew']
        we1, be1 = p['edge1']
        we2, be2 = p['edge2']
        wn1, bn1 = p['node1']
        wn2, bn2 = p['node2']

        w1p = jnp.pad(w1[:, 2 * H:].T, ((0, 5), (0, 0)))
        w2t = jnp.pad(w2.T, ((0, 0), (0, 5)))
        eww = jnp.pad(we, ((0, 0), (0, 5)))
        msc = jnp.stack([p['scale'][0], be[0]]).reshape(1, 2)
        msc = jnp.pad(msc, ((0, 0), (0, 6)))
        we1p = jnp.pad(we1[:, 2 * H:].T, ((0, 5), (0, 0)))

        xg, cr, cc = _sc_gather4(h, coords, row, col)
        delta = _edge_a_call(xg, cr, cc,
                             w1[:, :2 * H].T, w1p,
                             _r(b1, 2 * H), w2t, eww, msc)
        q = _sc_scatter8(delta, row, zeros8)
        coords = _cupd_call(coords, q[0], q[1])
        cr2, cc2 = _sc_gather2(coords, row, col)
        e = _edge_b_call(xg, cr2, cc2,
                         we1[:, :2 * H].T, we1p,
                         _r(be1, 2 * H), we2.T, _r(be2, H))
        ag = _sc_scatter64(e, row, zeros64)
        h = _node_call(h, ag[0], ag[1],
                       wn1[:, :H].T, wn1[:, H:].T, _r(bn1, 2 * H),
                       wn2.T, _r(bn2, H))

    wg1, bg1 = params['gate1']
    wg2, bg2 = params['gate2']
    wg3, bg3 = params['gate3']
    wo, bo = params['output_proj']
    g3t = jnp.pad(wg3.T, ((0, 0), (0, 7)))
    g3b = jnp.pad(bg3.reshape(1, 1), ((0, 0), (0, 7)))
    return _head_call(h, bcol, wg1.T, _r(bg1, H),
                      _r(params['ln_g'], H), _r(params['ln_b'], H),
                      wg2.T, _r(bg2, H // 2), g3t, g3b,
                      wo.T, _r(bo, OUT))


# speedup vs baseline: 2.1058x; 1.2946x over previous
import functools

import jax
import jax.numpy as jnp
from jax import lax
from jax.experimental import pallas as pl
from jax.experimental.pallas import tpu as pltpu
from jax.experimental.pallas import tpu_sc as plsc

N = 10000
E = 320000
D_IN = 128
H = 64
OUT = 128
NPAD = 10240

NC = 2
NS = 16
NW = NC * NS
EPW = E // NW
K = 80
NIT = EPW // K
RPT = NPAD // NS

_mesh = plsc.VectorSubcoreMesh(core_axis_name="c", subcore_axis_name="s")
_sc_params = pltpu.CompilerParams(use_tc_tiling_on_sc=False)

f32 = jnp.float32
i32 = jnp.int32


@functools.partial(
    pl.kernel,
    out_type=(
        jax.ShapeDtypeStruct((E, 2 * H), f32),
        jax.ShapeDtypeStruct((E // 8, 128), f32),
    ),
    mesh=_mesh,
    compiler_params=_sc_params,
    scratch_types=[
        pltpu.VMEM((K,), i32),
        pltpu.VMEM((K,), i32),
        pltpu.VMEM((K, H), f32),
        pltpu.VMEM((K, H), f32),
        pltpu.VMEM((K, 2 * H), f32),
        pltpu.VMEM((K, 16), f32),
        pltpu.VMEM((K, 16), f32),
        pltpu.VMEM((K // 8, 128), f32),
        pltpu.SemaphoreType.DMA,
    ],
)
def _sc_gather4(h_hbm, c_hbm, row_hbm, col_hbm,
                xg_hbm, rel_hbm,
                ir, ic, bxr, bxc, bxg, bcr, bcc, brel, sem):
    wid = lax.axis_index("s") * NC + lax.axis_index("c")
    base = wid * EPW
    L = 16

    def body(j, carry):
        off = base + j * K
        pltpu.sync_copy(row_hbm.at[pl.ds(off, K)], ir)
        pltpu.sync_copy(col_hbm.at[pl.ds(off, K)], ic)
        c1 = pltpu.async_copy(h_hbm.at[ir], bxr, sem)
        c2 = pltpu.async_copy(h_hbm.at[ic], bxc, sem)
        c3 = pltpu.async_copy(c_hbm.at[ir], bcr, sem)
        c4 = pltpu.async_copy(c_hbm.at[ic], bcc, sem)
        c1.wait(); c2.wait(); c3.wait(); c4.wait()

        def ilv(r, carry2):
            for q in range(H // L):
                bxg[r, pl.ds(L * q, L)] = bxr[r, pl.ds(L * q, L)]
                bxg[r, pl.ds(H + L * q, L)] = bxc[r, pl.ds(L * q, L)]
            brel[r // 8, pl.ds(L * (r % 8), L)] = bcr[r, :] - bcc[r, :]
            return carry2

        lax.fori_loop(0, K, ilv, 0)
        pltpu.sync_copy(bxg, xg_hbm.at[pl.ds(off, K)])
        pltpu.sync_copy(brel, rel_hbm.at[pl.ds(off // 8, K // 8)])
        return carry

    lax.fori_loop(0, NIT, body, 0)


@functools.partial(
    pl.kernel,
    out_type=jax.ShapeDtypeStruct((E // 8, 128), f32),
    mesh=_mesh,
    compiler_params=_sc_params,
    scratch_types=[
        pltpu.VMEM((K,), i32),
        pltpu.VMEM((K,), i32),
        pltpu.VMEM((K, 16), f32),
        pltpu.VMEM((K, 16), f32),
        pltpu.VMEM((K // 8, 128), f32),
        pltpu.SemaphoreType.DMA,
    ],
)
def _sc_gather2(c_hbm, row_hbm, col_hbm, rel_hbm,
                ir, ic, bcr, bcc, brel, sem):
    wid = lax.axis_index("s") * NC + lax.axis_index("c")
    base = wid * EPW
    L = 16

    def body(j, carry):
        off = base + j * K
        pltpu.sync_copy(row_hbm.at[pl.ds(off, K)], ir)
        pltpu.sync_copy(col_hbm.at[pl.ds(off, K)], ic)
        c1 = pltpu.async_copy(c_hbm.at[ir], bcr, sem)
        c2 = pltpu.async_copy(c_hbm.at[ic], bcc, sem)
        c1.wait(); c2.wait()

        def sub(r, carry2):
            brel[r // 8, pl.ds(L * (r % 8), L)] = bcr[r, :] - bcc[r, :]
            return carry2

        lax.fori_loop(0, K, sub, 0)
        pltpu.sync_copy(brel, rel_hbm.at[pl.ds(off // 8, K // 8)])
        return carry

    lax.fori_loop(0, NIT, body, 0)


def _make_sc_scatter(W):
    @functools.partial(
        pl.kernel,
        out_type=jax.ShapeDtypeStruct((2, NPAD, W), f32),
        mesh=_mesh,
        compiler_params=_sc_params,
        scratch_types=[
            pltpu.VMEM((K,), i32),
            pltpu.VMEM((K, W), f32),
            pltpu.VMEM_SHARED((NPAD, W), f32),
            pltpu.SemaphoreType.DMA,
        ],
    )
    def scat(val_hbm, row_hbm, zero_hbm, out_hbm, idx, vbuf, acc, sem):
        cid = lax.axis_index("c")
        sid = lax.axis_index("s")
        wid = sid * NC + cid
        pltpu.sync_copy(zero_hbm.at[pl.ds(sid * RPT, RPT)],
                        acc.at[pl.ds(sid * RPT, RPT)])
        plsc.subcore_barrier()
        base = wid * EPW

        def body(j, carry):
            off = base + j * K
            pltpu.sync_copy(row_hbm.at[pl.ds(off, K)], idx)
            pltpu.sync_copy(val_hbm.at[pl.ds(off, K)], vbuf)
            pltpu.async_copy(vbuf, acc.at[idx], sem, add=True).wait()
            return carry

        lax.fori_loop(0, NIT, body, 0)
        plsc.subcore_barrier()
        pltpu.sync_copy(acc.at[pl.ds(sid * RPT, RPT)],
                        out_hbm.at[cid, pl.ds(sid * RPT, RPT)])

    return scat


_sc_scatter16 = _make_sc_scatter(16)
_sc_scatter64 = _make_sc_scatter(H)


EB = 512
NBK = 1024


def _full(shape):
    return pl.BlockSpec(shape, lambda *_: tuple(0 for _ in shape))


def _rows(bs, w):
    return pl.BlockSpec((bs, w), lambda i, *_: (i, 0))


def _proj_body(x_ref, wt, b, o_ref):
    o_ref[...] = (jnp.dot(x_ref[...], wt[...], preferred_element_type=f32)
                  + b[...])


def _edge_a_body(xg, relp, w1x, wxp, b1, w2b, mones, wz, msc, dout):
    urel = jnp.dot(relp[...], wxp[...],
                   preferred_element_type=f32).reshape(EB, 2 * H)
    u = jnp.dot(xg[...], w1x[...], preferred_element_type=f32) + urel + b1[...]
    u = u * jax.nn.sigmoid(u)
    up = u.reshape(EB // 8, 8 * 2 * H)
    dp = jnp.dot(up, w2b[...], preferred_element_type=f32)
    n2 = jnp.dot(dp * dp, mones[...], preferred_element_type=f32)
    nrm = jnp.maximum(jnp.sqrt(n2), 1e-8)
    ew = jax.nn.sigmoid(jnp.dot(relp[...], wz[...], preferred_element_type=f32)
                        + msc[0, 1])
    dout[...] = dp * ((msc[0, 0] * ew) / nrm)


def _edge_b_body(xg, relp, e1x, wxp, b1, e2b, b2p, eout):
    urel = jnp.dot(relp[...], wxp[...],
                   preferred_element_type=f32).reshape(EB, 2 * H)
    u = jnp.dot(xg[...], e1x[...], preferred_element_type=f32) + urel + b1[...]
    u = u * jax.nn.sigmoid(u)
    up = u.reshape(EB // 2, 2 * 2 * H)
    v = jnp.dot(up, e2b[...], preferred_element_type=f32) + b2p[...]
    eout[...] = v * jax.nn.sigmoid(v)


def _cupd_body(c, q0, q1, cout):
    cout[...] = c[...] + q0[...] + q1[...]


def _node_body(h, p0, p1, n1h, n1a, b1, n2t, b2, hout):
    agg = p0[...] + p1[...]
    u = (jnp.dot(h[...], n1h[...], preferred_element_type=f32)
         + jnp.dot(agg, n1a[...], preferred_element_type=f32) + b1[...])
    u = u * jax.nn.sigmoid(u)
    hout[...] = h[...] + jnp.dot(u, n2t[...], preferred_element_type=f32) + b2[...]


def _head_body(h, bcol, g1t, g1b, lng, lnb, g2t, g2b, g3t, g3b, wot, bo,
               out, m_ref, d_ref, p_ref):
    ph = pl.program_id(0)
    i = pl.program_id(1)
    G = pl.num_programs(1)

    @pl.when(jnp.logical_and(ph == 0, i == 0))
    def _():
        m_ref[...] = jnp.full((1, 8), -1e30, f32)

    @pl.when(jnp.logical_and(ph == 1, i == 0))
    def _():
        d_ref[...] = jnp.zeros((8, 8), f32)
        p_ref[...] = jnp.zeros((8, H), f32)

    g = jnp.dot(h[...], g1t[...], preferred_element_type=f32) + g1b[...]
    mu = jnp.mean(g, axis=-1, keepdims=True)
    var = jnp.mean((g - mu) ** 2, axis=-1, keepdims=True)
    g = (g - mu) * lax.rsqrt(var + 1e-5) * lng[...] + lnb[...]
    g = jnp.maximum(g, 0.0)
    q = jnp.maximum(jnp.dot(g, g2t[...], preferred_element_type=f32)
                    + g2b[...], 0.0)
    s = jnp.dot(q, g3t[...], preferred_element_type=f32) + g3b[...]
    s0 = s[:, 0:1]
    oh = bcol[...] == lax.broadcasted_iota(i32, (1, 8), 1)

    @pl.when(ph == 0)
    def _():
        sm = jnp.where(oh, s0, -1e30)
        bm = jnp.max(sm, axis=0, keepdims=True)
        m_ref[...] = jnp.maximum(m_ref[...], bm)

    @pl.when(ph == 1)
    def _():
        w = jnp.where(oh, jnp.exp(s0 - m_ref[...]), 0.0)
        dn = (((0,), (0,)), ((), ()))
        d_ref[...] += lax.dot_general(w, jnp.ones((NBK, 8), f32), dn,
                                      preferred_element_type=f32)
        p_ref[...] += lax.dot_general(w, h[...], dn,
                                      preferred_element_type=f32)

    @pl.when(jnp.logical_and(ph == 1, i == G - 1))
    def _():
        pooled = p_ref[...] / d_ref[:, 0:1]
        out[...] = (jnp.dot(pooled, wot[...], preferred_element_type=f32)
                    + bo[...])


def _proj_call(xp, wt, b):
    return pl.pallas_call(
        _proj_body,
        grid=(NPAD // NBK,),
        in_specs=[_rows(NBK, D_IN), _full((D_IN, H)), _full((1, H))],
        out_specs=_rows(NBK, H),
        out_shape=jax.ShapeDtypeStruct((NPAD, H), f32),
    )(xp, wt, b)


def _edge_a_call(xg, relp, w1x, wxp, b1, w2b, mones, wz, msc):
    return pl.pallas_call(
        _edge_a_body,
        grid=(E // EB,),
        in_specs=[_rows(EB, 2 * H), _rows(EB // 8, 128),
                  _full((2 * H, 2 * H)), _full((128, 8 * 2 * H)),
                  _full((1, 2 * H)), _full((8 * 2 * H, 128)),
                  _full((128, 128)), _full((128, 128)),
                  pl.BlockSpec(memory_space=pltpu.SMEM)],
        out_specs=_rows(EB // 8, 128),
        out_shape=jax.ShapeDtypeStruct((E // 8, 128), f32),
    )(xg, relp, w1x, wxp, b1, w2b, mones, wz, msc)


def _edge_b_call(xg, relp, e1x, wxp, b1, e2b, b2p):
    return pl.pallas_call(
        _edge_b_body,
        grid=(E // EB,),
        in_specs=[_rows(EB, 2 * H), _rows(EB // 8, 128),
                  _full((2 * H, 2 * H)), _full((128, 8 * 2 * H)),
                  _full((1, 2 * H)), _full((2 * 2 * H, 128)), _full((1, 128))],
        out_specs=_rows(EB // 2, 128),
        out_shape=jax.ShapeDtypeStruct((E // 2, 128), f32),
    )(xg, relp, e1x, wxp, b1, e2b, b2p)


def _cupd_call(c, q0, q1):
    return pl.pallas_call(
        _cupd_body,
        grid=(NPAD // NBK,),
        in_specs=[_rows(NBK, 16)] * 3,
        out_specs=_rows(NBK, 16),
        out_shape=jax.ShapeDtypeStruct((NPAD, 16), f32),
    )(c, q0, q1)


def _node_call(h, p0, p1, n1h, n1a, b1, n2t, b2):
    return pl.pallas_call(
        _node_body,
        grid=(NPAD // NBK,),
        in_specs=[_rows(NBK, H), _rows(NBK, H), _rows(NBK, H),
                  _full((H, 2 * H)), _full((H, 2 * H)), _full((1, 2 * H)),
                  _full((2 * H, H)), _full((1, H))],
        out_specs=_rows(NBK, H),
        out_shape=jax.ShapeDtypeStruct((NPAD, H), f32),
    )(h, p0, p1, n1h, n1a, b1, n2t, b2)


def _head_call(h, bcol, g1t, g1b, lng, lnb, g2t, g2b, g3t, g3b, wot, bo):
    return pl.pallas_call(
        _head_body,
        grid=(2, NPAD // NBK),
        in_specs=[pl.BlockSpec((NBK, H), lambda p, i: (i, 0)),
                  pl.BlockSpec((NBK, 1), lambda p, i: (i, 0)),
                  _full((H, H)), _full((1, H)), _full((1, H)), _full((1, H)),
                  _full((H, H // 2)), _full((1, H // 2)),
                  _full((H // 2, 8)), _full((1, 8)),
                  _full((H, OUT)), _full((1, OUT))],
        out_specs=pl.BlockSpec((8, OUT), lambda p, i: (0, 0)),
        out_shape=jax.ShapeDtypeStruct((8, OUT), f32),
        scratch_shapes=[pltpu.VMEM((1, 8), f32), pltpu.VMEM((8, 8), f32),
                        pltpu.VMEM((8, H), f32)],
    )(h, bcol, g1t, g1b, lng, lnb, g2t, g2b, g3t, g3b, wot, bo)


def _r(v, n):
    return v.reshape(1, n)


def _expand_rel(w3):
    wp = jnp.pad(w3, ((0, 13), (0, 0)))
    return jnp.einsum('qp,tc->qtpc', jnp.eye(8, dtype=f32),
                      wp).reshape(128, 8 * 128)


def _blockdiag(wt, k):
    a, b = wt.shape
    return jnp.einsum('pq,dc->pdqc', jnp.eye(k, dtype=f32),
                      wt).reshape(k * a, k * b)


def kernel(x, pos, edge_index, batch, params):
    row = edge_index[0].astype(i32)
    col = edge_index[1].astype(i32)

    xp = jnp.pad(x, ((0, NPAD - N), (0, 0)))
    coords = jnp.pad(pos, ((0, NPAD - N), (0, 13)))
    bcol = jnp.pad(batch.astype(i32), (0, NPAD - N),
                   constant_values=127).reshape(NPAD, 1)
    zeros16 = jnp.zeros((NPAD, 16), f32)
    zeros64 = jnp.zeros((NPAD, H), f32)

    wi, bi = params['input_proj']
    h = _proj_call(xp, wi.T, _r(bi, H))

    for p in params['layers']:
        w1, b1 = p['coord1']
        w2, b2 = p['coord2']
        we, be = p['ew']
        we1, be1 = p['edge1']
        we2, be2 = p['edge2']
        wn1, bn1 = p['node1']
        wn2, bn2 = p['node2']

        wxp_a = _expand_rel(w1[:, 2 * H:].T)
        wxp_b = _expand_rel(we1[:, 2 * H:].T)
        w2b = _blockdiag(jnp.pad(w2.T, ((0, 0), (0, 13))), 8)
        e2b = _blockdiag(we2.T, 2)
        b2p = jnp.tile(be2, 2).reshape(1, 128)
        ones16 = jnp.ones((16,), f32)
        mones = jnp.einsum('qp,t,c->qtpc', jnp.eye(8, dtype=f32),
                           ones16, ones16).reshape(128, 128)
        wz = jnp.einsum('qp,t,c->qtpc', jnp.eye(8, dtype=f32),
                        jnp.pad(we[0], (0, 13)), ones16).reshape(128, 128)
        msc = jnp.stack([p['scale'][0], be[0]]).reshape(1, 2)
        msc = jnp.pad(msc, ((0, 0), (0, 6)))

        xg, relp = _sc_gather4(h, coords, row, col)
        delta = _edge_a_call(xg, relp,
                             w1[:, :2 * H].T, wxp_a,
                             _r(b1, 2 * H), w2b, mones, wz, msc)
        q = _sc_scatter16(delta.reshape(E, 16), row, zeros16)
        coords = _cupd_call(coords, q[0], q[1])
        relp2 = _sc_gather2(coords, row, col)
        e = _edge_b_call(xg, relp2,
                         we1[:, :2 * H].T, wxp_b,
                         _r(be1, 2 * H), e2b, b2p)
        ag = _sc_scatter64(e.reshape(E, H), row, zeros64)
        h = _node_call(h, ag[0], ag[1],
                       wn1[:, :H].T, wn1[:, H:].T, _r(bn1, 2 * H),
                       wn2.T, _r(bn2, H))

    wg1, bg1 = params['gate1']
    wg2, bg2 = params['gate2']
    wg3, bg3 = params['gate3']
    wo, bo = params['output_proj']
    g3t = jnp.pad(wg3.T, ((0, 0), (0, 7)))
    g3b = jnp.pad(bg3.reshape(1, 1), ((0, 0), (0, 7)))
    return _head_call(h, bcol, wg1.T, _r(bg1, H),
                      _r(params['ln_g'], H), _r(params['ln_b'], H),
                      wg2.T, _r(bg2, H // 2), g3t, g3b,
                      wo.T, _r(bo, OUT))


# speedup vs baseline: 2.9127x; 1.3832x over previous
import functools

import jax
import jax.numpy as jnp
from jax import lax
from jax.experimental import pallas as pl
from jax.experimental.pallas import tpu as pltpu
from jax.experimental.pallas import tpu_sc as plsc

N = 10000
E = 320000
D_IN = 128
H = 64
OUT = 128
NPAD = 10240

NC = 2
NS = 16
NW = NC * NS
EPW = E // NW
K = 80
NIT = EPW // K
RPT = NPAD // NS

_mesh = plsc.VectorSubcoreMesh(core_axis_name="c", subcore_axis_name="s")
_sc_params = pltpu.CompilerParams(use_tc_tiling_on_sc=False)

f32 = jnp.float32
i32 = jnp.int32


@functools.partial(
    pl.kernel,
    out_type=(
        jax.ShapeDtypeStruct((E, 2 * H), f32),
        jax.ShapeDtypeStruct((E // 8, 128), f32),
    ),
    mesh=_mesh,
    compiler_params=_sc_params,
    scratch_types=[
        pltpu.VMEM((K,), i32),
        pltpu.VMEM((K,), i32),
        pltpu.VMEM((K,), i32),
        pltpu.VMEM((K,), i32),
        pltpu.VMEM((K, H), f32),
        pltpu.VMEM((K, H), f32),
        pltpu.VMEM((K, H), f32),
        pltpu.VMEM((K, H), f32),
        pltpu.VMEM((K, 16), f32),
        pltpu.VMEM((K, 16), f32),
        pltpu.VMEM((K, 16), f32),
        pltpu.VMEM((K, 16), f32),
        pltpu.VMEM((K // 8, 128), f32),
        pltpu.SemaphoreType.DMA,
        pltpu.SemaphoreType.DMA,
    ],
)
def _sc_gather4(h_hbm, c_hbm, row_hbm, col_hbm,
                xg_hbm, rel_hbm,
                ir0, ic0, ir1, ic1, bxr0, bxc0, bxr1, bxc1,
                bcr0, bcc0, bcr1, bcc1, brel, sem_g, sem_w):
    wid = lax.axis_index("s") * NC + lax.axis_index("c")
    base = wid * EPW
    L = 16

    def pack(bcr, bcc):
        def sub(r, carry2):
            brel[r // 8, pl.ds(L * (r % 8), L)] = bcr[r, :] - bcc[r, :]
            return carry2

        lax.fori_loop(0, K, sub, 0)

    def step(j, ir, ic, bxr, bxc, bcr, bcc,
             pir, pic, pxr, pxc, pcr, pcc):
        off = base + j * K
        g1 = pltpu.async_copy(h_hbm.at[ir], bxr, sem_g)
        g2 = pltpu.async_copy(h_hbm.at[ic], bxc, sem_g)
        g3 = pltpu.async_copy(c_hbm.at[ir], bcr, sem_g)
        g4 = pltpu.async_copy(c_hbm.at[ic], bcc, sem_g)

        @pl.when(j + 1 < NIT)
        def _():
            pltpu.sync_copy(row_hbm.at[pl.ds(off + K, K)], pir)
            pltpu.sync_copy(col_hbm.at[pl.ds(off + K, K)], pic)

        @pl.when(j > 0)
        def _():
            pack(pcr, pcc)
            w1 = pltpu.async_copy(pxr, xg_hbm.at[pl.ds(off - K, K),
                                                 pl.ds(0, H)], sem_w)
            w2 = pltpu.async_copy(pxc, xg_hbm.at[pl.ds(off - K, K),
                                                 pl.ds(H, H)], sem_w)
            w3 = pltpu.async_copy(brel, rel_hbm.at[pl.ds((off - K) // 8,
                                                         K // 8)], sem_w)
            w1.wait(); w2.wait(); w3.wait()

        g1.wait(); g2.wait(); g3.wait(); g4.wait()

    pltpu.sync_copy(row_hbm.at[pl.ds(base, K)], ir0)
    pltpu.sync_copy(col_hbm.at[pl.ds(base, K)], ic0)

    def body(j, carry):
        @pl.when(j % 2 == 0)
        def _():
            step(j, ir0, ic0, bxr0, bxc0, bcr0, bcc0,
                 ir1, ic1, bxr1, bxc1, bcr1, bcc1)

        @pl.when(j % 2 == 1)
        def _():
            step(j, ir1, ic1, bxr1, bxc1, bcr1, bcc1,
                 ir0, ic0, bxr0, bxc0, bcr0, bcc0)

        return carry

    lax.fori_loop(0, NIT, body, 0)
    offl = base + (NIT - 1) * K
    pack(bcr0, bcc0)
    pltpu.sync_copy(bxr0, xg_hbm.at[pl.ds(offl, K), pl.ds(0, H)])
    pltpu.sync_copy(bxc0, xg_hbm.at[pl.ds(offl, K), pl.ds(H, H)])
    pltpu.sync_copy(brel, rel_hbm.at[pl.ds(offl // 8, K // 8)])


@functools.partial(
    pl.kernel,
    out_type=jax.ShapeDtypeStruct((E // 8, 128), f32),
    mesh=_mesh,
    compiler_params=_sc_params,
    scratch_types=[
        pltpu.VMEM((K,), i32),
        pltpu.VMEM((K,), i32),
        pltpu.VMEM((K,), i32),
        pltpu.VMEM((K,), i32),
        pltpu.VMEM((K, 16), f32),
        pltpu.VMEM((K, 16), f32),
        pltpu.VMEM((K, 16), f32),
        pltpu.VMEM((K, 16), f32),
        pltpu.VMEM((K // 8, 128), f32),
        pltpu.SemaphoreType.DMA,
        pltpu.SemaphoreType.DMA,
    ],
)
def _sc_gather2(c_hbm, row_hbm, col_hbm, rel_hbm,
                ir0, ic0, ir1, ic1, bcr0, bcc0, bcr1, bcc1,
                brel, sem_g, sem_w):
    wid = lax.axis_index("s") * NC + lax.axis_index("c")
    base = wid * EPW
    L = 16

    def pack(bcr, bcc):
        def sub(r, carry2):
            brel[r // 8, pl.ds(L * (r % 8), L)] = bcr[r, :] - bcc[r, :]
            return carry2

        lax.fori_loop(0, K, sub, 0)

    def step(j, ir, ic, bcr, bcc, pir, pic, pcr, pcc):
        off = base + j * K
        g1 = pltpu.async_copy(c_hbm.at[ir], bcr, sem_g)
        g2 = pltpu.async_copy(c_hbm.at[ic], bcc, sem_g)

        @pl.when(j + 1 < NIT)
        def _():
            pltpu.sync_copy(row_hbm.at[pl.ds(off + K, K)], pir)
            pltpu.sync_copy(col_hbm.at[pl.ds(off + K, K)], pic)

        @pl.when(j > 0)
        def _():
            pack(pcr, pcc)
            w3 = pltpu.async_copy(brel, rel_hbm.at[pl.ds((off - K) // 8,
                                                         K // 8)], sem_w)
            w3.wait()

        g1.wait(); g2.wait()

    pltpu.sync_copy(row_hbm.at[pl.ds(base, K)], ir0)
    pltpu.sync_copy(col_hbm.at[pl.ds(base, K)], ic0)

    def body(j, carry):
        @pl.when(j % 2 == 0)
        def _():
            step(j, ir0, ic0, bcr0, bcc0, ir1, ic1, bcr1, bcc1)

        @pl.when(j % 2 == 1)
        def _():
            step(j, ir1, ic1, bcr1, bcc1, ir0, ic0, bcr0, bcc0)

        return carry

    lax.fori_loop(0, NIT, body, 0)
    offl = base + (NIT - 1) * K
    pack(bcr0, bcc0)
    pltpu.sync_copy(brel, rel_hbm.at[pl.ds(offl // 8, K // 8)])


def _make_sc_scatter(W):
    @functools.partial(
        pl.kernel,
        out_type=jax.ShapeDtypeStruct((2, NPAD, W), f32),
        mesh=_mesh,
        compiler_params=_sc_params,
        scratch_types=[
            pltpu.VMEM((K,), i32),
            pltpu.VMEM((K,), i32),
            pltpu.VMEM((K, W), f32),
            pltpu.VMEM((K, W), f32),
            pltpu.VMEM_SHARED((NPAD, W), f32),
            pltpu.SemaphoreType.DMA,
            pltpu.SemaphoreType.DMA,
        ],
    )
    def scat(val_hbm, row_hbm, zero_hbm, out_hbm,
             i0, i1, v0, v1, acc, sem_s, sem_v):
        cid = lax.axis_index("c")
        sid = lax.axis_index("s")
        wid = sid * NC + cid
        pltpu.sync_copy(zero_hbm.at[pl.ds(sid * RPT, RPT)],
                        acc.at[pl.ds(sid * RPT, RPT)])
        plsc.subcore_barrier()
        base = wid * EPW

        def step(j, ci, cv, pi, pv):
            s = pltpu.async_copy(cv, acc.at[ci], sem_s, add=True)

            @pl.when(j + 1 < NIT)
            def _():
                off2 = base + (j + 1) * K
                l1 = pltpu.async_copy(row_hbm.at[pl.ds(off2, K)], pi, sem_v)
                l2 = pltpu.async_copy(val_hbm.at[pl.ds(off2, K)], pv, sem_v)
                l1.wait(); l2.wait()

            s.wait()

        pltpu.sync_copy(row_hbm.at[pl.ds(base, K)], i0)
        pltpu.sync_copy(val_hbm.at[pl.ds(base, K)], v0)

        def body(j, carry):
            @pl.when(j % 2 == 0)
            def _():
                step(j, i0, v0, i1, v1)

            @pl.when(j % 2 == 1)
            def _():
                step(j, i1, v1, i0, v0)

            return carry

        lax.fori_loop(0, NIT, body, 0)
        plsc.subcore_barrier()
        pltpu.sync_copy(acc.at[pl.ds(sid * RPT, RPT)],
                        out_hbm.at[cid, pl.ds(sid * RPT, RPT)])

    return scat


_sc_scatter16 = _make_sc_scatter(16)
_sc_scatter64 = _make_sc_scatter(H)


EB = 512
NBK = 1024


def _full(shape):
    return pl.BlockSpec(shape, lambda *_: tuple(0 for _ in shape))


def _rows(bs, w):
    return pl.BlockSpec((bs, w), lambda i, *_: (i, 0))


def _proj_body(x_ref, wt, b, o_ref):
    o_ref[...] = (jnp.dot(x_ref[...], wt[...], preferred_element_type=f32)
                  + b[...])


def _edge_a_body(xg, relp, w1x, wxp, b1, w2b, mones, wz, msc, dout):
    urel = jnp.dot(relp[...], wxp[...],
                   preferred_element_type=f32).reshape(EB, 2 * H)
    u = jnp.dot(xg[...], w1x[...], preferred_element_type=f32) + urel + b1[...]
    u = u * jax.nn.sigmoid(u)
    up = u.reshape(EB // 8, 8 * 2 * H)
    dp = jnp.dot(up, w2b[...], preferred_element_type=f32)
    n2 = jnp.dot(dp * dp, mones[...], preferred_element_type=f32)
    nrm = jnp.maximum(jnp.sqrt(n2), 1e-8)
    ew = jax.nn.sigmoid(jnp.dot(relp[...], wz[...], preferred_element_type=f32)
                        + msc[0, 1])
    dout[...] = dp * ((msc[0, 0] * ew) / nrm)


def _edge_b_body(xg, relp, e1x, wxp, b1, e2b, b2p, eout):
    urel = jnp.dot(relp[...], wxp[...],
                   preferred_element_type=f32).reshape(EB, 2 * H)
    u = jnp.dot(xg[...], e1x[...], preferred_element_type=f32) + urel + b1[...]
    u = u * jax.nn.sigmoid(u)
    up = u.reshape(EB // 2, 2 * 2 * H)
    v = jnp.dot(up, e2b[...], preferred_element_type=f32) + b2p[...]
    eout[...] = v * jax.nn.sigmoid(v)


def _cupd_body(c, q0, q1, cout):
    cout[...] = c[...] + q0[...] + q1[...]


def _node_body(h, p0, p1, n1h, n1a, b1, n2t, b2, hout):
    agg = p0[...] + p1[...]
    u = (jnp.dot(h[...], n1h[...], preferred_element_type=f32)
         + jnp.dot(agg, n1a[...], preferred_element_type=f32) + b1[...])
    u = u * jax.nn.sigmoid(u)
    hout[...] = h[...] + jnp.dot(u, n2t[...], preferred_element_type=f32) + b2[...]


def _head_body(h, bcol, g1t, g1b, lng, lnb, g2t, g2b, g3t, g3b, wot, bo,
               out, m_ref, d_ref, p_ref):
    ph = pl.program_id(0)
    i = pl.program_id(1)
    G = pl.num_programs(1)

    @pl.when(jnp.logical_and(ph == 0, i == 0))
    def _():
        m_ref[...] = jnp.full((1, 8), -1e30, f32)

    @pl.when(jnp.logical_and(ph == 1, i == 0))
    def _():
        d_ref[...] = jnp.zeros((8, 8), f32)
        p_ref[...] = jnp.zeros((8, H), f32)

    g = jnp.dot(h[...], g1t[...], preferred_element_type=f32) + g1b[...]
    mu = jnp.mean(g, axis=-1, keepdims=True)
    var = jnp.mean((g - mu) ** 2, axis=-1, keepdims=True)
    g = (g - mu) * lax.rsqrt(var + 1e-5) * lng[...] + lnb[...]
    g = jnp.maximum(g, 0.0)
    q = jnp.maximum(jnp.dot(g, g2t[...], preferred_element_type=f32)
                    + g2b[...], 0.0)
    s = jnp.dot(q, g3t[...], preferred_element_type=f32) + g3b[...]
    s0 = s[:, 0:1]
    oh = bcol[...] == lax.broadcasted_iota(i32, (1, 8), 1)

    @pl.when(ph == 0)
    def _():
        sm = jnp.where(oh, s0, -1e30)
        bm = jnp.max(sm, axis=0, keepdims=True)
        m_ref[...] = jnp.maximum(m_ref[...], bm)

    @pl.when(ph == 1)
    def _():
        w = jnp.where(oh, jnp.exp(s0 - m_ref[...]), 0.0)
        dn = (((0,), (0,)), ((), ()))
        d_ref[...] += lax.dot_general(w, jnp.ones((NBK, 8), f32), dn,
                                      preferred_element_type=f32)
        p_ref[...] += lax.dot_general(w, h[...], dn,
                                      preferred_element_type=f32)

    @pl.when(jnp.logical_and(ph == 1, i == G - 1))
    def _():
        pooled = p_ref[...] / d_ref[:, 0:1]
        out[...] = (jnp.dot(pooled, wot[...], preferred_element_type=f32)
                    + bo[...])


def _proj_call(xp, wt, b):
    return pl.pallas_call(
        _proj_body,
        grid=(NPAD // NBK,),
        in_specs=[_rows(NBK, D_IN), _full((D_IN, H)), _full((1, H))],
        out_specs=_rows(NBK, H),
        out_shape=jax.ShapeDtypeStruct((NPAD, H), f32),
    )(xp, wt, b)


def _edge_a_call(xg, relp, w1x, wxp, b1, w2b, mones, wz, msc):
    return pl.pallas_call(
        _edge_a_body,
        grid=(E // EB,),
        in_specs=[_rows(EB, 2 * H), _rows(EB // 8, 128),
                  _full((2 * H, 2 * H)), _full((128, 8 * 2 * H)),
                  _full((1, 2 * H)), _full((8 * 2 * H, 128)),
                  _full((128, 128)), _full((128, 128)),
                  pl.BlockSpec(memory_space=pltpu.SMEM)],
        out_specs=_rows(EB // 8, 128),
        out_shape=jax.ShapeDtypeStruct((E // 8, 128), f32),
    )(xg, relp, w1x, wxp, b1, w2b, mones, wz, msc)


def _edge_b_call(xg, relp, e1x, wxp, b1, e2b, b2p):
    return pl.pallas_call(
        _edge_b_body,
        grid=(E // EB,),
        in_specs=[_rows(EB, 2 * H), _rows(EB // 8, 128),
                  _full((2 * H, 2 * H)), _full((128, 8 * 2 * H)),
                  _full((1, 2 * H)), _full((2 * 2 * H, 128)), _full((1, 128))],
        out_specs=_rows(EB // 2, 128),
        out_shape=jax.ShapeDtypeStruct((E // 2, 128), f32),
    )(xg, relp, e1x, wxp, b1, e2b, b2p)


def _cupd_call(c, q0, q1):
    return pl.pallas_call(
        _cupd_body,
        grid=(NPAD // NBK,),
        in_specs=[_rows(NBK, 16)] * 3,
        out_specs=_rows(NBK, 16),
        out_shape=jax.ShapeDtypeStruct((NPAD, 16), f32),
    )(c, q0, q1)


def _node_call(h, p0, p1, n1h, n1a, b1, n2t, b2):
    return pl.pallas_call(
        _node_body,
        grid=(NPAD // NBK,),
        in_specs=[_rows(NBK, H), _rows(NBK, H), _rows(NBK, H),
                  _full((H, 2 * H)), _full((H, 2 * H)), _full((1, 2 * H)),
                  _full((2 * H, H)), _full((1, H))],
        out_specs=_rows(NBK, H),
        out_shape=jax.ShapeDtypeStruct((NPAD, H), f32),
    )(h, p0, p1, n1h, n1a, b1, n2t, b2)


def _head_call(h, bcol, g1t, g1b, lng, lnb, g2t, g2b, g3t, g3b, wot, bo):
    return pl.pallas_call(
        _head_body,
        grid=(2, NPAD // NBK),
        in_specs=[pl.BlockSpec((NBK, H), lambda p, i: (i, 0)),
                  pl.BlockSpec((NBK, 1), lambda p, i: (i, 0)),
                  _full((H, H)), _full((1, H)), _full((1, H)), _full((1, H)),
                  _full((H, H // 2)), _full((1, H // 2)),
                  _full((H // 2, 8)), _full((1, 8)),
                  _full((H, OUT)), _full((1, OUT))],
        out_specs=pl.BlockSpec((8, OUT), lambda p, i: (0, 0)),
        out_shape=jax.ShapeDtypeStruct((8, OUT), f32),
        scratch_shapes=[pltpu.VMEM((1, 8), f32), pltpu.VMEM((8, 8), f32),
                        pltpu.VMEM((8, H), f32)],
    )(h, bcol, g1t, g1b, lng, lnb, g2t, g2b, g3t, g3b, wot, bo)


def _r(v, n):
    return v.reshape(1, n)


def _expand_rel(w3):
    wp = jnp.pad(w3, ((0, 13), (0, 0)))
    return jnp.einsum('qp,tc->qtpc', jnp.eye(8, dtype=f32),
                      wp).reshape(128, 8 * 128)


def _blockdiag(wt, k):
    a, b = wt.shape
    return jnp.einsum('pq,dc->pdqc', jnp.eye(k, dtype=f32),
                      wt).reshape(k * a, k * b)


def kernel(x, pos, edge_index, batch, params):
    row = edge_index[0].astype(i32)
    col = edge_index[1].astype(i32)

    xp = jnp.pad(x, ((0, NPAD - N), (0, 0)))
    coords = jnp.pad(pos, ((0, NPAD - N), (0, 13)))
    bcol = jnp.pad(batch.astype(i32), (0, NPAD - N),
                   constant_values=127).reshape(NPAD, 1)
    zeros16 = jnp.zeros((NPAD, 16), f32)
    zeros64 = jnp.zeros((NPAD, H), f32)

    wi, bi = params['input_proj']
    h = _proj_call(xp, wi.T, _r(bi, H))

    for p in params['layers']:
        w1, b1 = p['coord1']
        w2, b2 = p['coord2']
        we, be = p['ew']
        we1, be1 = p['edge1']
        we2, be2 = p['edge2']
        wn1, bn1 = p['node1']
        wn2, bn2 = p['node2']

        wxp_a = _expand_rel(w1[:, 2 * H:].T)
        wxp_b = _expand_rel(we1[:, 2 * H:].T)
        w2b = _blockdiag(jnp.pad(w2.T, ((0, 0), (0, 13))), 8)
        e2b = _blockdiag(we2.T, 2)
        b2p = jnp.tile(be2, 2).reshape(1, 128)
        ones16 = jnp.ones((16,), f32)
        mones = jnp.einsum('qp,t,c->qtpc', jnp.eye(8, dtype=f32),
                           ones16, ones16).reshape(128, 128)
        wz = jnp.einsum('qp,t,c->qtpc', jnp.eye(8, dtype=f32),
                        jnp.pad(we[0], (0, 13)), ones16).reshape(128, 128)
        msc = jnp.stack([p['scale'][0], be[0]]).reshape(1, 2)
        msc = jnp.pad(msc, ((0, 0), (0, 6)))

        xg, relp = _sc_gather4(h, coords, row, col)
        delta = _edge_a_call(xg, relp,
                             w1[:, :2 * H].T, wxp_a,
                             _r(b1, 2 * H), w2b, mones, wz, msc)
        q = _sc_scatter16(delta.reshape(E, 16), row, zeros16)
        coords = _cupd_call(coords, q[0], q[1])
        relp2 = _sc_gather2(coords, row, col)
        e = _edge_b_call(xg, relp2,
                         we1[:, :2 * H].T, wxp_b,
                         _r(be1, 2 * H), e2b, b2p)
        ag = _sc_scatter64(e.reshape(E, H), row, zeros64)
        h = _node_call(h, ag[0], ag[1],
                       wn1[:, :H].T, wn1[:, H:].T, _r(bn1, 2 * H),
                       wn2.T, _r(bn2, H))

    wg1, bg1 = params['gate1']
    wg2, bg2 = params['gate2']
    wg3, bg3 = params['gate3']
    wo, bo = params['output_proj']
    g3t = jnp.pad(wg3.T, ((0, 0), (0, 7)))
    g3b = jnp.pad(bg3.reshape(1, 1), ((0, 0), (0, 7)))
    return _head_call(h, bcol, wg1.T, _r(bg1, H),
                      _r(params['ln_g'], H), _r(params['ln_b'], H),
                      wg2.T, _r(bg2, H // 2), g3t, g3b,
                      wo.T, _r(bo, OUT))


# speedup vs baseline: 3.9824x; 1.3672x over previous
import functools

import jax
import jax.numpy as jnp
from jax import lax
from jax.experimental import pallas as pl
from jax.experimental.pallas import tpu as pltpu
from jax.experimental.pallas import tpu_sc as plsc

N = 10000
E = 320000
D_IN = 128
H = 64
OUT = 128
NPAD = 10240

NC = 2
NS = 16
NW = NC * NS
EPW = E // NW
K = 80
NIT = EPW // K
RPT = NPAD // NS

_mesh = plsc.VectorSubcoreMesh(core_axis_name="c", subcore_axis_name="s")
_sc_params = pltpu.CompilerParams(use_tc_tiling_on_sc=False)

f32 = jnp.float32
i32 = jnp.int32


@functools.partial(
    pl.kernel,
    out_type=(
        jax.ShapeDtypeStruct((E, 2 * H), f32),
        jax.ShapeDtypeStruct((E // 8, 128), f32),
    ),
    mesh=_mesh,
    compiler_params=_sc_params,
    scratch_types=[
        pltpu.VMEM((K,), i32),
        pltpu.VMEM((K,), i32),
        pltpu.VMEM((K,), i32),
        pltpu.VMEM((K,), i32),
        pltpu.VMEM((K, H), f32),
        pltpu.VMEM((K, H), f32),
        pltpu.VMEM((K, H), f32),
        pltpu.VMEM((K, H), f32),
        pltpu.VMEM((K, 16), f32),
        pltpu.VMEM((K, 16), f32),
        pltpu.VMEM((K, 16), f32),
        pltpu.VMEM((K, 16), f32),
        pltpu.VMEM((K // 8, 128), f32),
        pltpu.SemaphoreType.DMA,
        pltpu.SemaphoreType.DMA,
    ],
)
def _sc_gather4(h_hbm, c_hbm, row_hbm, col_hbm,
                xg_hbm, rel_hbm,
                ir0, ic0, ir1, ic1, bxr0, bxc0, bxr1, bxc1,
                bcr0, bcc0, bcr1, bcc1, brel, sem_g, sem_w):
    wid = lax.axis_index("s") * NC + lax.axis_index("c")
    base = wid * EPW
    L = 16

    def pack(bcr, bcc):
        def sub(r, carry2):
            brel[r // 8, pl.ds(L * (r % 8), L)] = bcr[r, :] - bcc[r, :]
            return carry2

        lax.fori_loop(0, K, sub, 0)

    def step(j, ir, ic, bxr, bxc, bcr, bcc,
             pir, pic, pxr, pxc, pcr, pcc):
        off = base + j * K
        g1 = pltpu.async_copy(h_hbm.at[ir], bxr, sem_g)
        g2 = pltpu.async_copy(h_hbm.at[ic], bxc, sem_g)
        g3 = pltpu.async_copy(c_hbm.at[ir], bcr, sem_g)
        g4 = pltpu.async_copy(c_hbm.at[ic], bcc, sem_g)

        @pl.when(j + 1 < NIT)
        def _():
            pltpu.sync_copy(row_hbm.at[pl.ds(off + K, K)], pir)
            pltpu.sync_copy(col_hbm.at[pl.ds(off + K, K)], pic)

        @pl.when(j > 0)
        def _():
            pack(pcr, pcc)
            w1 = pltpu.async_copy(pxr, xg_hbm.at[pl.ds(off - K, K),
                                                 pl.ds(0, H)], sem_w)
            w2 = pltpu.async_copy(pxc, xg_hbm.at[pl.ds(off - K, K),
                                                 pl.ds(H, H)], sem_w)
            w3 = pltpu.async_copy(brel, rel_hbm.at[pl.ds((off - K) // 8,
                                                         K // 8)], sem_w)
            w1.wait(); w2.wait(); w3.wait()

        g1.wait(); g2.wait(); g3.wait(); g4.wait()

    pltpu.sync_copy(row_hbm.at[pl.ds(base, K)], ir0)
    pltpu.sync_copy(col_hbm.at[pl.ds(base, K)], ic0)

    def body(j, carry):
        @pl.when(j % 2 == 0)
        def _():
            step(j, ir0, ic0, bxr0, bxc0, bcr0, bcc0,
                 ir1, ic1, bxr1, bxc1, bcr1, bcc1)

        @pl.when(j % 2 == 1)
        def _():
            step(j, ir1, ic1, bxr1, bxc1, bcr1, bcc1,
                 ir0, ic0, bxr0, bxc0, bcr0, bcc0)

        return carry

    lax.fori_loop(0, NIT, body, 0)
    offl = base + (NIT - 1) * K
    pack(bcr0, bcc0)
    pltpu.sync_copy(bxr0, xg_hbm.at[pl.ds(offl, K), pl.ds(0, H)])
    pltpu.sync_copy(bxc0, xg_hbm.at[pl.ds(offl, K), pl.ds(H, H)])
    pltpu.sync_copy(brel, rel_hbm.at[pl.ds(offl // 8, K // 8)])


@functools.partial(
    pl.kernel,
    out_type=jax.ShapeDtypeStruct((E // 8, 128), f32),
    mesh=_mesh,
    compiler_params=_sc_params,
    scratch_types=[
        pltpu.VMEM((K,), i32),
        pltpu.VMEM((K,), i32),
        pltpu.VMEM((K,), i32),
        pltpu.VMEM((K,), i32),
        pltpu.VMEM((K, 16), f32),
        pltpu.VMEM((K, 16), f32),
        pltpu.VMEM((K, 16), f32),
        pltpu.VMEM((K, 16), f32),
        pltpu.VMEM((K // 8, 128), f32),
        pltpu.SemaphoreType.DMA,
        pltpu.SemaphoreType.DMA,
    ],
)
def _sc_gather2(c_hbm, row_hbm, col_hbm, rel_hbm,
                ir0, ic0, ir1, ic1, bcr0, bcc0, bcr1, bcc1,
                brel, sem_g, sem_w):
    wid = lax.axis_index("s") * NC + lax.axis_index("c")
    base = wid * EPW
    L = 16

    def pack(bcr, bcc):
        def sub(r, carry2):
            brel[r // 8, pl.ds(L * (r % 8), L)] = bcr[r, :] - bcc[r, :]
            return carry2

        lax.fori_loop(0, K, sub, 0)

    def step(j, ir, ic, bcr, bcc, pir, pic, pcr, pcc):
        off = base + j * K
        g1 = pltpu.async_copy(c_hbm.at[ir], bcr, sem_g)
        g2 = pltpu.async_copy(c_hbm.at[ic], bcc, sem_g)

        @pl.when(j + 1 < NIT)
        def _():
            pltpu.sync_copy(row_hbm.at[pl.ds(off + K, K)], pir)
            pltpu.sync_copy(col_hbm.at[pl.ds(off + K, K)], pic)

        @pl.when(j > 0)
        def _():
            pack(pcr, pcc)
            w3 = pltpu.async_copy(brel, rel_hbm.at[pl.ds((off - K) // 8,
                                                         K // 8)], sem_w)
            w3.wait()

        g1.wait(); g2.wait()

    pltpu.sync_copy(row_hbm.at[pl.ds(base, K)], ir0)
    pltpu.sync_copy(col_hbm.at[pl.ds(base, K)], ic0)

    def body(j, carry):
        @pl.when(j % 2 == 0)
        def _():
            step(j, ir0, ic0, bcr0, bcc0, ir1, ic1, bcr1, bcc1)

        @pl.when(j % 2 == 1)
        def _():
            step(j, ir1, ic1, bcr1, bcc1, ir0, ic0, bcr0, bcc0)

        return carry

    lax.fori_loop(0, NIT, body, 0)
    offl = base + (NIT - 1) * K
    pack(bcr0, bcc0)
    pltpu.sync_copy(brel, rel_hbm.at[pl.ds(offl // 8, K // 8)])


def _make_sc_scatter(W):
    @functools.partial(
        pl.kernel,
        out_type=jax.ShapeDtypeStruct((2, NPAD, W), f32),
        mesh=_mesh,
        compiler_params=_sc_params,
        scratch_types=[
            pltpu.VMEM((K,), i32),
            pltpu.VMEM((K,), i32),
            pltpu.VMEM((K, W), f32),
            pltpu.VMEM((K, W), f32),
            pltpu.VMEM_SHARED((NPAD, W), f32),
            pltpu.SemaphoreType.DMA,
            pltpu.SemaphoreType.DMA,
        ],
    )
    def scat(val_hbm, row_hbm, zero_hbm, out_hbm,
             i0, i1, v0, v1, acc, sem_s, sem_v):
        cid = lax.axis_index("c")
        sid = lax.axis_index("s")
        wid = sid * NC + cid
        pltpu.sync_copy(zero_hbm.at[pl.ds(sid * RPT, RPT)],
                        acc.at[pl.ds(sid * RPT, RPT)])
        plsc.subcore_barrier()
        base = wid * EPW

        def step(j, ci, cv, pi, pv):
            s = pltpu.async_copy(cv, acc.at[ci], sem_s, add=True)

            @pl.when(j + 1 < NIT)
            def _():
                off2 = base + (j + 1) * K
                l1 = pltpu.async_copy(row_hbm.at[pl.ds(off2, K)], pi, sem_v)
                l2 = pltpu.async_copy(val_hbm.at[pl.ds(off2, K)], pv, sem_v)
                l1.wait(); l2.wait()

            s.wait()

        pltpu.sync_copy(row_hbm.at[pl.ds(base, K)], i0)
        pltpu.sync_copy(val_hbm.at[pl.ds(base, K)], v0)

        def body(j, carry):
            @pl.when(j % 2 == 0)
            def _():
                step(j, i0, v0, i1, v1)

            @pl.when(j % 2 == 1)
            def _():
                step(j, i1, v1, i0, v0)

            return carry

        lax.fori_loop(0, NIT, body, 0)
        plsc.subcore_barrier()
        pltpu.sync_copy(acc.at[pl.ds(sid * RPT, RPT)],
                        out_hbm.at[cid, pl.ds(sid * RPT, RPT)])

    return scat


_sc_scatter16 = _make_sc_scatter(16)
_sc_scatter64 = _make_sc_scatter(H)


EB = 1280
NBK = 1024


def _full(shape):
    return pl.BlockSpec(shape, lambda *_: tuple(0 for _ in shape))


def _rows(bs, w):
    return pl.BlockSpec((bs, w), lambda i, *_: (i, 0))


def _proj_body(x_ref, wt, b, o_ref):
    o_ref[...] = (jnp.dot(x_ref[...], wt[...], preferred_element_type=f32)
                  + b[...])


def _edge_a_body(xg, relp, w1x, wxp, b1, w2b, mones, wz, msc, dout):
    urel = jnp.dot(relp[...], wxp[...],
                   preferred_element_type=f32).reshape(EB, 2 * H)
    u = jnp.dot(xg[...], w1x[...], preferred_element_type=f32) + urel + b1[...]
    u = u * jax.nn.sigmoid(u)
    up = u.reshape(EB // 8, 8 * 2 * H)
    dp = jnp.dot(up, w2b[...], preferred_element_type=f32)
    n2 = jnp.dot(dp * dp, mones[...], preferred_element_type=f32)
    nrm = jnp.maximum(jnp.sqrt(n2), 1e-8)
    ew = jax.nn.sigmoid(jnp.dot(relp[...], wz[...], preferred_element_type=f32)
                        + msc[0, 1])
    dout[...] = dp * ((msc[0, 0] * ew) / nrm)


def _edge_b_body(xg, relp, e1x, wxp, b1, e2b, b2p, eout):
    urel = jnp.dot(relp[...], wxp[...],
                   preferred_element_type=f32).reshape(EB, 2 * H)
    u = jnp.dot(xg[...], e1x[...], preferred_element_type=f32) + urel + b1[...]
    u = u * jax.nn.sigmoid(u)
    up = u.reshape(EB // 2, 2 * 2 * H)
    v = jnp.dot(up, e2b[...], preferred_element_type=f32) + b2p[...]
    eout[...] = v * jax.nn.sigmoid(v)


def _cupd_body(c, q0, q1, cout):
    cout[...] = c[...] + q0[...] + q1[...]


def _node_body(h, p0, p1, n1h, n1a, b1, n2t, b2, hout):
    agg = p0[...] + p1[...]
    u = (jnp.dot(h[...], n1h[...], preferred_element_type=f32)
         + jnp.dot(agg, n1a[...], preferred_element_type=f32) + b1[...])
    u = u * jax.nn.sigmoid(u)
    hout[...] = h[...] + jnp.dot(u, n2t[...], preferred_element_type=f32) + b2[...]


def _head_body(h, bcol, g1t, g1b, lng, lnb, g2t, g2b, g3t, g3b, wot, bo,
               out, m_ref, d_ref, p_ref):
    ph = pl.program_id(0)
    i = pl.program_id(1)
    G = pl.num_programs(1)

    @pl.when(jnp.logical_and(ph == 0, i == 0))
    def _():
        m_ref[...] = jnp.full((1, 8), -1e30, f32)

    @pl.when(jnp.logical_and(ph == 1, i == 0))
    def _():
        d_ref[...] = jnp.zeros((8, 8), f32)
        p_ref[...] = jnp.zeros((8, H), f32)

    g = jnp.dot(h[...], g1t[...], preferred_element_type=f32) + g1b[...]
    mu = jnp.mean(g, axis=-1, keepdims=True)
    var = jnp.mean((g - mu) ** 2, axis=-1, keepdims=True)
    g = (g - mu) * lax.rsqrt(var + 1e-5) * lng[...] + lnb[...]
    g = jnp.maximum(g, 0.0)
    q = jnp.maximum(jnp.dot(g, g2t[...], preferred_element_type=f32)
                    + g2b[...], 0.0)
    s = jnp.dot(q, g3t[...], preferred_element_type=f32) + g3b[...]
    s0 = s[:, 0:1]
    oh = bcol[...] == lax.broadcasted_iota(i32, (1, 8), 1)

    @pl.when(ph == 0)
    def _():
        sm = jnp.where(oh, s0, -1e30)
        bm = jnp.max(sm, axis=0, keepdims=True)
        m_ref[...] = jnp.maximum(m_ref[...], bm)

    @pl.when(ph == 1)
    def _():
        w = jnp.where(oh, jnp.exp(s0 - m_ref[...]), 0.0)
        dn = (((0,), (0,)), ((), ()))
        d_ref[...] += lax.dot_general(w, jnp.ones((NBK, 8), f32), dn,
                                      preferred_element_type=f32)
        p_ref[...] += lax.dot_general(w, h[...], dn,
                                      preferred_element_type=f32)

    @pl.when(jnp.logical_and(ph == 1, i == G - 1))
    def _():
        pooled = p_ref[...] / d_ref[:, 0:1]
        out[...] = (jnp.dot(pooled, wot[...], preferred_element_type=f32)
                    + bo[...])


def _proj_call(xp, wt, b):
    return pl.pallas_call(
        _proj_body,
        grid=(NPAD // NBK,),
        in_specs=[_rows(NBK, D_IN), _full((D_IN, H)), _full((1, H))],
        out_specs=_rows(NBK, H),
        out_shape=jax.ShapeDtypeStruct((NPAD, H), f32),
    )(xp, wt, b)


def _edge_a_call(xg, relp, w1x, wxp, b1, w2b, mones, wz, msc):
    return pl.pallas_call(
        _edge_a_body,
        grid=(E // EB,),
        in_specs=[_rows(EB, 2 * H), _rows(EB // 8, 128),
                  _full((2 * H, 2 * H)), _full((128, 8 * 2 * H)),
                  _full((1, 2 * H)), _full((8 * 2 * H, 128)),
                  _full((128, 128)), _full((128, 128)),
                  pl.BlockSpec(memory_space=pltpu.SMEM)],
        out_specs=_rows(EB // 8, 128),
        out_shape=jax.ShapeDtypeStruct((E // 8, 128), f32),
    )(xg, relp, w1x, wxp, b1, w2b, mones, wz, msc)


def _edge_b_call(xg, relp, e1x, wxp, b1, e2b, b2p):
    return pl.pallas_call(
        _edge_b_body,
        grid=(E // EB,),
        in_specs=[_rows(EB, 2 * H), _rows(EB // 8, 128),
                  _full((2 * H, 2 * H)), _full((128, 8 * 2 * H)),
                  _full((1, 2 * H)), _full((2 * 2 * H, 128)), _full((1, 128))],
        out_specs=_rows(EB // 2, 128),
        out_shape=jax.ShapeDtypeStruct((E // 2, 128), f32),
    )(xg, relp, e1x, wxp, b1, e2b, b2p)


def _cupd_call(c, q0, q1):
    return pl.pallas_call(
        _cupd_body,
        grid=(NPAD // NBK,),
        in_specs=[_rows(NBK, 16)] * 3,
        out_specs=_rows(NBK, 16),
        out_shape=jax.ShapeDtypeStruct((NPAD, 16), f32),
    )(c, q0, q1)


def _node_call(h, p0, p1, n1h, n1a, b1, n2t, b2):
    return pl.pallas_call(
        _node_body,
        grid=(NPAD // NBK,),
        in_specs=[_rows(NBK, H), _rows(NBK, H), _rows(NBK, H),
                  _full((H, 2 * H)), _full((H, 2 * H)), _full((1, 2 * H)),
                  _full((2 * H, H)), _full((1, H))],
        out_specs=_rows(NBK, H),
        out_shape=jax.ShapeDtypeStruct((NPAD, H), f32),
    )(h, p0, p1, n1h, n1a, b1, n2t, b2)


def _head_call(h, bcol, g1t, g1b, lng, lnb, g2t, g2b, g3t, g3b, wot, bo):
    return pl.pallas_call(
        _head_body,
        grid=(2, NPAD // NBK),
        in_specs=[pl.BlockSpec((NBK, H), lambda p, i: (i, 0)),
                  pl.BlockSpec((NBK, 1), lambda p, i: (i, 0)),
                  _full((H, H)), _full((1, H)), _full((1, H)), _full((1, H)),
                  _full((H, H // 2)), _full((1, H // 2)),
                  _full((H // 2, 8)), _full((1, 8)),
                  _full((H, OUT)), _full((1, OUT))],
        out_specs=pl.BlockSpec((8, OUT), lambda p, i: (0, 0)),
        out_shape=jax.ShapeDtypeStruct((8, OUT), f32),
        scratch_shapes=[pltpu.VMEM((1, 8), f32), pltpu.VMEM((8, 8), f32),
                        pltpu.VMEM((8, H), f32)],
    )(h, bcol, g1t, g1b, lng, lnb, g2t, g2b, g3t, g3b, wot, bo)


def _r(v, n):
    return v.reshape(1, n)


def _expand_rel(w3):
    wp = jnp.pad(w3, ((0, 13), (0, 0)))
    return jnp.einsum('qp,tc->qtpc', jnp.eye(8, dtype=f32),
                      wp).reshape(128, 8 * 128)


def _blockdiag(wt, k):
    a, b = wt.shape
    return jnp.einsum('pq,dc->pdqc', jnp.eye(k, dtype=f32),
                      wt).reshape(k * a, k * b)


def kernel(x, pos, edge_index, batch, params):
    row = edge_index[0].astype(i32)
    col = edge_index[1].astype(i32)

    xp = jnp.pad(x, ((0, NPAD - N), (0, 0)))
    coords = jnp.pad(pos, ((0, NPAD - N), (0, 13)))
    bcol = jnp.pad(batch.astype(i32), (0, NPAD - N),
                   constant_values=127).reshape(NPAD, 1)
    zeros16 = jnp.zeros((NPAD, 16), f32)
    zeros64 = jnp.zeros((NPAD, H), f32)

    wi, bi = params['input_proj']
    h = _proj_call(xp, wi.T, _r(bi, H))

    for p in params['layers']:
        w1, b1 = p['coord1']
        w2, b2 = p['coord2']
        we, be = p['ew']
        we1, be1 = p['edge1']
        we2, be2 = p['edge2']
        wn1, bn1 = p['node1']
        wn2, bn2 = p['node2']

        wxp_a = _expand_rel(w1[:, 2 * H:].T)
        wxp_b = _expand_rel(we1[:, 2 * H:].T)
        w2b = _blockdiag(jnp.pad(w2.T, ((0, 0), (0, 13))), 8)
        e2b = _blockdiag(we2.T, 2)
        b2p = jnp.tile(be2, 2).reshape(1, 128)
        ones16 = jnp.ones((16,), f32)
        mones = jnp.einsum('qp,t,c->qtpc', jnp.eye(8, dtype=f32),
                           ones16, ones16).reshape(128, 128)
        wz = jnp.einsum('qp,t,c->qtpc', jnp.eye(8, dtype=f32),
                        jnp.pad(we[0], (0, 13)), ones16).reshape(128, 128)
        msc = jnp.stack([p['scale'][0], be[0]]).reshape(1, 2)
        msc = jnp.pad(msc, ((0, 0), (0, 6)))

        xg, relp = _sc_gather4(h, coords, row, col)
        delta = _edge_a_call(xg, relp,
                             w1[:, :2 * H].T, wxp_a,
                             _r(b1, 2 * H), w2b, mones, wz, msc)
        q = _sc_scatter16(delta.reshape(E, 16), row, zeros16)
        coords = _cupd_call(coords, q[0], q[1])
        relp2 = _sc_gather2(coords, row, col)
        e = _edge_b_call(xg, relp2,
                         we1[:, :2 * H].T, wxp_b,
                         _r(be1, 2 * H), e2b, b2p)
        ag = _sc_scatter64(e.reshape(E, H), row, zeros64)
        h = _node_call(h, ag[0], ag[1],
                       wn1[:, :H].T, wn1[:, H:].T, _r(bn1, 2 * H),
                       wn2.T, _r(bn2, H))

    wg1, bg1 = params['gate1']
    wg2, bg2 = params['gate2']
    wg3, bg3 = params['gate3']
    wo, bo = params['output_proj']
    g3t = jnp.pad(wg3.T, ((0, 0), (0, 7)))
    g3b = jnp.pad(bg3.reshape(1, 1), ((0, 0), (0, 7)))
    return _head_call(h, bcol, wg1.T, _r(bg1, H),
                      _r(params['ln_g'], H), _r(params['ln_b'], H),
                      wg2.T, _r(bg2, H // 2), g3t, g3b,
                      wo.T, _r(bo, OUT))


# speedup vs baseline: 3.9909x; 1.0021x over previous
import functools

import jax
import jax.numpy as jnp
from jax import lax
from jax.experimental import pallas as pl
from jax.experimental.pallas import tpu as pltpu
from jax.experimental.pallas import tpu_sc as plsc

N = 10000
E = 320000
D_IN = 128
H = 64
OUT = 128
NPAD = 10240

NC = 2
NS = 16
NW = NC * NS
EPW = E // NW
K = 80
NIT = EPW // K
RPT = NPAD // NS

_mesh = plsc.VectorSubcoreMesh(core_axis_name="c", subcore_axis_name="s")
_sc_params = pltpu.CompilerParams(use_tc_tiling_on_sc=False)

f32 = jnp.float32
i32 = jnp.int32


@functools.partial(
    pl.kernel,
    out_type=(
        jax.ShapeDtypeStruct((E, 2 * H), f32),
        jax.ShapeDtypeStruct((E // 8, 128), f32),
    ),
    mesh=_mesh,
    compiler_params=_sc_params,
    scratch_types=[
        pltpu.VMEM((K,), i32),
        pltpu.VMEM((K,), i32),
        pltpu.VMEM((K,), i32),
        pltpu.VMEM((K,), i32),
        pltpu.VMEM((K, H), f32),
        pltpu.VMEM((K, H), f32),
        pltpu.VMEM((K, H), f32),
        pltpu.VMEM((K, H), f32),
        pltpu.VMEM((K, 16), f32),
        pltpu.VMEM((K, 16), f32),
        pltpu.VMEM((K, 16), f32),
        pltpu.VMEM((K, 16), f32),
        pltpu.VMEM((K // 8, 128), f32),
        pltpu.SemaphoreType.DMA,
        pltpu.SemaphoreType.DMA,
    ],
)
def _sc_gather4(h_hbm, c_hbm, row_hbm, col_hbm,
                xg_hbm, rel_hbm,
                ir0, ic0, ir1, ic1, bxr0, bxc0, bxr1, bxc1,
                bcr0, bcc0, bcr1, bcc1, brel, sem_g, sem_w):
    wid = lax.axis_index("s") * NC + lax.axis_index("c")
    base = wid * EPW
    L = 16

    def pack(bcr, bcc):
        def sub(r, carry2):
            brel[r // 8, pl.ds(L * (r % 8), L)] = bcr[r, :] - bcc[r, :]
            return carry2

        lax.fori_loop(0, K, sub, 0)

    def step(j, ir, ic, bxr, bxc, bcr, bcc,
             pir, pic, pxr, pxc, pcr, pcc):
        off = base + j * K
        g1 = pltpu.async_copy(h_hbm.at[ir], bxr, sem_g)
        g2 = pltpu.async_copy(h_hbm.at[ic], bxc, sem_g)
        g3 = pltpu.async_copy(c_hbm.at[ir], bcr, sem_g)
        g4 = pltpu.async_copy(c_hbm.at[ic], bcc, sem_g)

        @pl.when(j + 1 < NIT)
        def _():
            pltpu.sync_copy(row_hbm.at[pl.ds(off + K, K)], pir)
            pltpu.sync_copy(col_hbm.at[pl.ds(off + K, K)], pic)

        @pl.when(j > 0)
        def _():
            pack(pcr, pcc)
            pltpu.async_copy(pxr, xg_hbm.at[pl.ds(off - K, K),
                                            pl.ds(0, H)], sem_w)
            pltpu.async_copy(pxc, xg_hbm.at[pl.ds(off - K, K),
                                            pl.ds(H, H)], sem_w)
            pltpu.async_copy(brel, rel_hbm.at[pl.ds((off - K) // 8,
                                              K // 8)], sem_w)

        g1.wait(); g2.wait(); g3.wait(); g4.wait()

        @pl.when(j > 0)
        def _():
            dw1 = pltpu.make_async_copy(pxr, xg_hbm.at[pl.ds(off - K, K),
                                                       pl.ds(0, H)], sem_w)
            dw2 = pltpu.make_async_copy(pxc, xg_hbm.at[pl.ds(off - K, K),
                                                       pl.ds(H, H)], sem_w)
            dw3 = pltpu.make_async_copy(brel, rel_hbm.at[pl.ds((off - K) // 8,
                                                         K // 8)], sem_w)
            dw1.wait(); dw2.wait(); dw3.wait()

    pltpu.sync_copy(row_hbm.at[pl.ds(base, K)], ir0)
    pltpu.sync_copy(col_hbm.at[pl.ds(base, K)], ic0)

    def body(j, carry):
        @pl.when(j % 2 == 0)
        def _():
            step(j, ir0, ic0, bxr0, bxc0, bcr0, bcc0,
                 ir1, ic1, bxr1, bxc1, bcr1, bcc1)

        @pl.when(j % 2 == 1)
        def _():
            step(j, ir1, ic1, bxr1, bxc1, bcr1, bcc1,
                 ir0, ic0, bxr0, bxc0, bcr0, bcc0)

        return carry

    lax.fori_loop(0, NIT, body, 0)
    offl = base + (NIT - 1) * K
    pack(bcr0, bcc0)
    pltpu.sync_copy(bxr0, xg_hbm.at[pl.ds(offl, K), pl.ds(0, H)])
    pltpu.sync_copy(bxc0, xg_hbm.at[pl.ds(offl, K), pl.ds(H, H)])
    pltpu.sync_copy(brel, rel_hbm.at[pl.ds(offl // 8, K // 8)])


@functools.partial(
    pl.kernel,
    out_type=jax.ShapeDtypeStruct((E // 8, 128), f32),
    mesh=_mesh,
    compiler_params=_sc_params,
    scratch_types=[
        pltpu.VMEM((K,), i32),
        pltpu.VMEM((K,), i32),
        pltpu.VMEM((K,), i32),
        pltpu.VMEM((K,), i32),
        pltpu.VMEM((K, 16), f32),
        pltpu.VMEM((K, 16), f32),
        pltpu.VMEM((K, 16), f32),
        pltpu.VMEM((K, 16), f32),
        pltpu.VMEM((K // 8, 128), f32),
        pltpu.SemaphoreType.DMA,
        pltpu.SemaphoreType.DMA,
    ],
)
def _sc_gather2(c_hbm, row_hbm, col_hbm, rel_hbm,
                ir0, ic0, ir1, ic1, bcr0, bcc0, bcr1, bcc1,
                brel, sem_g, sem_w):
    wid = lax.axis_index("s") * NC + lax.axis_index("c")
    base = wid * EPW
    L = 16

    def pack(bcr, bcc):
        def sub(r, carry2):
            brel[r // 8, pl.ds(L * (r % 8), L)] = bcr[r, :] - bcc[r, :]
            return carry2

        lax.fori_loop(0, K, sub, 0)

    def step(j, ir, ic, bcr, bcc, pir, pic, pcr, pcc):
        off = base + j * K
        g1 = pltpu.async_copy(c_hbm.at[ir], bcr, sem_g)
        g2 = pltpu.async_copy(c_hbm.at[ic], bcc, sem_g)

        @pl.when(j + 1 < NIT)
        def _():
            pltpu.sync_copy(row_hbm.at[pl.ds(off + K, K)], pir)
            pltpu.sync_copy(col_hbm.at[pl.ds(off + K, K)], pic)

        @pl.when(j > 0)
        def _():
            pack(pcr, pcc)
            pltpu.async_copy(brel, rel_hbm.at[pl.ds((off - K) // 8,
                                              K // 8)], sem_w)

        g1.wait(); g2.wait()

        @pl.when(j > 0)
        def _():
            pltpu.make_async_copy(brel, rel_hbm.at[pl.ds((off - K) // 8,
                                                   K // 8)], sem_w).wait()

    pltpu.sync_copy(row_hbm.at[pl.ds(base, K)], ir0)
    pltpu.sync_copy(col_hbm.at[pl.ds(base, K)], ic0)

    def body(j, carry):
        @pl.when(j % 2 == 0)
        def _():
            step(j, ir0, ic0, bcr0, bcc0, ir1, ic1, bcr1, bcc1)

        @pl.when(j % 2 == 1)
        def _():
            step(j, ir1, ic1, bcr1, bcc1, ir0, ic0, bcr0, bcc0)

        return carry

    lax.fori_loop(0, NIT, body, 0)
    offl = base + (NIT - 1) * K
    pack(bcr0, bcc0)
    pltpu.sync_copy(brel, rel_hbm.at[pl.ds(offl // 8, K // 8)])


def _make_sc_scatter(W):
    @functools.partial(
        pl.kernel,
        out_type=jax.ShapeDtypeStruct((2, NPAD, W), f32),
        mesh=_mesh,
        compiler_params=_sc_params,
        scratch_types=[
            pltpu.VMEM((K,), i32),
            pltpu.VMEM((K,), i32),
            pltpu.VMEM((K, W), f32),
            pltpu.VMEM((K, W), f32),
            pltpu.VMEM_SHARED((NPAD, W), f32),
            pltpu.SemaphoreType.DMA,
            pltpu.SemaphoreType.DMA,
        ],
    )
    def scat(val_hbm, row_hbm, zero_hbm, out_hbm,
             i0, i1, v0, v1, acc, sem_s, sem_v):
        cid = lax.axis_index("c")
        sid = lax.axis_index("s")
        wid = sid * NC + cid
        pltpu.sync_copy(zero_hbm.at[pl.ds(sid * RPT, RPT)],
                        acc.at[pl.ds(sid * RPT, RPT)])
        plsc.subcore_barrier()
        base = wid * EPW

        def step(j, ci, cv, pi, pv):
            s = pltpu.async_copy(cv, acc.at[ci], sem_s, add=True)

            @pl.when(j + 1 < NIT)
            def _():
                off2 = base + (j + 1) * K
                l1 = pltpu.async_copy(row_hbm.at[pl.ds(off2, K)], pi, sem_v)
                l2 = pltpu.async_copy(val_hbm.at[pl.ds(off2, K)], pv, sem_v)
                l1.wait(); l2.wait()

            s.wait()

        pltpu.sync_copy(row_hbm.at[pl.ds(base, K)], i0)
        pltpu.sync_copy(val_hbm.at[pl.ds(base, K)], v0)

        def body(j, carry):
            @pl.when(j % 2 == 0)
            def _():
                step(j, i0, v0, i1, v1)

            @pl.when(j % 2 == 1)
            def _():
                step(j, i1, v1, i0, v0)

            return carry

        lax.fori_loop(0, NIT, body, 0)
        plsc.subcore_barrier()
        pltpu.sync_copy(acc.at[pl.ds(sid * RPT, RPT)],
                        out_hbm.at[cid, pl.ds(sid * RPT, RPT)])

    return scat


_sc_scatter16 = _make_sc_scatter(16)
_sc_scatter64 = _make_sc_scatter(H)


EB = 1280
NBK = 1024


def _full(shape):
    return pl.BlockSpec(shape, lambda *_: tuple(0 for _ in shape))


def _rows(bs, w):
    return pl.BlockSpec((bs, w), lambda i, *_: (i, 0))


def _proj_body(x_ref, wt, b, o_ref):
    o_ref[...] = (jnp.dot(x_ref[...], wt[...], preferred_element_type=f32)
                  + b[...])


def _edge_a_body(xg, relp, w1x, wxp, b1, w2b, mones, wz, msc, dout):
    urel = jnp.dot(relp[...], wxp[...],
                   preferred_element_type=f32).reshape(EB, 2 * H)
    u = jnp.dot(xg[...], w1x[...], preferred_element_type=f32) + urel + b1[...]
    u = u * jax.nn.sigmoid(u)
    up = u.reshape(EB // 8, 8 * 2 * H)
    dp = jnp.dot(up, w2b[...], preferred_element_type=f32)
    n2 = jnp.dot(dp * dp, mones[...], preferred_element_type=f32)
    nrm = jnp.maximum(jnp.sqrt(n2), 1e-8)
    ew = jax.nn.sigmoid(jnp.dot(relp[...], wz[...], preferred_element_type=f32)
                        + msc[0, 1])
    dout[...] = dp * ((msc[0, 0] * ew) / nrm)


def _edge_b_body(xg, relp, e1x, wxp, b1, e2b, b2p, eout):
    urel = jnp.dot(relp[...], wxp[...],
                   preferred_element_type=f32).reshape(EB, 2 * H)
    u = jnp.dot(xg[...], e1x[...], preferred_element_type=f32) + urel + b1[...]
    u = u * jax.nn.sigmoid(u)
    up = u.reshape(EB // 2, 2 * 2 * H)
    v = jnp.dot(up, e2b[...], preferred_element_type=f32) + b2p[...]
    eout[...] = v * jax.nn.sigmoid(v)


def _cupd_body(c, q0, q1, cout):
    cout[...] = c[...] + q0[...] + q1[...]


def _node_body(h, p0, p1, n1h, n1a, b1, n2t, b2, hout):
    agg = p0[...] + p1[...]
    u = (jnp.dot(h[...], n1h[...], preferred_element_type=f32)
         + jnp.dot(agg, n1a[...], preferred_element_type=f32) + b1[...])
    u = u * jax.nn.sigmoid(u)
    hout[...] = h[...] + jnp.dot(u, n2t[...], preferred_element_type=f32) + b2[...]


def _head_body(h, bcol, g1t, g1b, lng, lnb, g2t, g2b, g3t, g3b, wot, bo,
               out, m_ref, d_ref, p_ref):
    ph = pl.program_id(0)
    i = pl.program_id(1)
    G = pl.num_programs(1)

    @pl.when(jnp.logical_and(ph == 0, i == 0))
    def _():
        m_ref[...] = jnp.full((1, 8), -1e30, f32)

    @pl.when(jnp.logical_and(ph == 1, i == 0))
    def _():
        d_ref[...] = jnp.zeros((8, 8), f32)
        p_ref[...] = jnp.zeros((8, H), f32)

    g = jnp.dot(h[...], g1t[...], preferred_element_type=f32) + g1b[...]
    mu = jnp.mean(g, axis=-1, keepdims=True)
    var = jnp.mean((g - mu) ** 2, axis=-1, keepdims=True)
    g = (g - mu) * lax.rsqrt(var + 1e-5) * lng[...] + lnb[...]
    g = jnp.maximum(g, 0.0)
    q = jnp.maximum(jnp.dot(g, g2t[...], preferred_element_type=f32)
                    + g2b[...], 0.0)
    s = jnp.dot(q, g3t[...], preferred_element_type=f32) + g3b[...]
    s0 = s[:, 0:1]
    oh = bcol[...] == lax.broadcasted_iota(i32, (1, 8), 1)

    @pl.when(ph == 0)
    def _():
        sm = jnp.where(oh, s0, -1e30)
        bm = jnp.max(sm, axis=0, keepdims=True)
        m_ref[...] = jnp.maximum(m_ref[...], bm)

    @pl.when(ph == 1)
    def _():
        w = jnp.where(oh, jnp.exp(s0 - m_ref[...]), 0.0)
        dn = (((0,), (0,)), ((), ()))
        d_ref[...] += lax.dot_general(w, jnp.ones((NBK, 8), f32), dn,
                                      preferred_element_type=f32)
        p_ref[...] += lax.dot_general(w, h[...], dn,
                                      preferred_element_type=f32)

    @pl.when(jnp.logical_and(ph == 1, i == G - 1))
    def _():
        pooled = p_ref[...] / d_ref[:, 0:1]
        out[...] = (jnp.dot(pooled, wot[...], preferred_element_type=f32)
                    + bo[...])


def _proj_call(xp, wt, b):
    return pl.pallas_call(
        _proj_body,
        grid=(NPAD // NBK,),
        in_specs=[_rows(NBK, D_IN), _full((D_IN, H)), _full((1, H))],
        out_specs=_rows(NBK, H),
        out_shape=jax.ShapeDtypeStruct((NPAD, H), f32),
    )(xp, wt, b)


def _edge_a_call(xg, relp, w1x, wxp, b1, w2b, mones, wz, msc):
    return pl.pallas_call(
        _edge_a_body,
        grid=(E // EB,),
        in_specs=[_rows(EB, 2 * H), _rows(EB // 8, 128),
                  _full((2 * H, 2 * H)), _full((128, 8 * 2 * H)),
                  _full((1, 2 * H)), _full((8 * 2 * H, 128)),
                  _full((128, 128)), _full((128, 128)),
                  pl.BlockSpec(memory_space=pltpu.SMEM)],
        out_specs=_rows(EB // 8, 128),
        out_shape=jax.ShapeDtypeStruct((E // 8, 128), f32),
    )(xg, relp, w1x, wxp, b1, w2b, mones, wz, msc)


def _edge_b_call(xg, relp, e1x, wxp, b1, e2b, b2p):
    return pl.pallas_call(
        _edge_b_body,
        grid=(E // EB,),
        in_specs=[_rows(EB, 2 * H), _rows(EB // 8, 128),
                  _full((2 * H, 2 * H)), _full((128, 8 * 2 * H)),
                  _full((1, 2 * H)), _full((2 * 2 * H, 128)), _full((1, 128))],
        out_specs=_rows(EB // 2, 128),
        out_shape=jax.ShapeDtypeStruct((E // 2, 128), f32),
    )(xg, relp, e1x, wxp, b1, e2b, b2p)


def _cupd_call(c, q0, q1):
    return pl.pallas_call(
        _cupd_body,
        grid=(NPAD // NBK,),
        in_specs=[_rows(NBK, 16)] * 3,
        out_specs=_rows(NBK, 16),
        out_shape=jax.ShapeDtypeStruct((NPAD, 16), f32),
    )(c, q0, q1)


def _node_call(h, p0, p1, n1h, n1a, b1, n2t, b2):
    return pl.pallas_call(
        _node_body,
        grid=(NPAD // NBK,),
        in_specs=[_rows(NBK, H), _rows(NBK, H), _rows(NBK, H),
                  _full((H, 2 * H)), _full((H, 2 * H)), _full((1, 2 * H)),
                  _full((2 * H, H)), _full((1, H))],
        out_specs=_rows(NBK, H),
        out_shape=jax.ShapeDtypeStruct((NPAD, H), f32),
    )(h, p0, p1, n1h, n1a, b1, n2t, b2)


def _head_call(h, bcol, g1t, g1b, lng, lnb, g2t, g2b, g3t, g3b, wot, bo):
    return pl.pallas_call(
        _head_body,
        grid=(2, NPAD // NBK),
        in_specs=[pl.BlockSpec((NBK, H), lambda p, i: (i, 0)),
                  pl.BlockSpec((NBK, 1), lambda p, i: (i, 0)),
                  _full((H, H)), _full((1, H)), _full((1, H)), _full((1, H)),
                  _full((H, H // 2)), _full((1, H // 2)),
                  _full((H // 2, 8)), _full((1, 8)),
                  _full((H, OUT)), _full((1, OUT))],
        out_specs=pl.BlockSpec((8, OUT), lambda p, i: (0, 0)),
        out_shape=jax.ShapeDtypeStruct((8, OUT), f32),
        scratch_shapes=[pltpu.VMEM((1, 8), f32), pltpu.VMEM((8, 8), f32),
                        pltpu.VMEM((8, H), f32)],
    )(h, bcol, g1t, g1b, lng, lnb, g2t, g2b, g3t, g3b, wot, bo)


def _r(v, n):
    return v.reshape(1, n)


def _expand_rel(w3):
    wp = jnp.pad(w3, ((0, 13), (0, 0)))
    return jnp.einsum('qp,tc->qtpc', jnp.eye(8, dtype=f32),
                      wp).reshape(128, 8 * 128)


def _blockdiag(wt, k):
    a, b = wt.shape
    return jnp.einsum('pq,dc->pdqc', jnp.eye(k, dtype=f32),
                      wt).reshape(k * a, k * b)


def kernel(x, pos, edge_index, batch, params):
    row = edge_index[0].astype(i32)
    col = edge_index[1].astype(i32)

    xp = jnp.pad(x, ((0, NPAD - N), (0, 0)))
    coords = jnp.pad(pos, ((0, NPAD - N), (0, 13)))
    bcol = jnp.pad(batch.astype(i32), (0, NPAD - N),
                   constant_values=127).reshape(NPAD, 1)
    zeros16 = jnp.zeros((NPAD, 16), f32)
    zeros64 = jnp.zeros((NPAD, H), f32)

    wi, bi = params['input_proj']
    h = _proj_call(xp, wi.T, _r(bi, H))

    for p in params['layers']:
        w1, b1 = p['coord1']
        w2, b2 = p['coord2']
        we, be = p['ew']
        we1, be1 = p['edge1']
        we2, be2 = p['edge2']
        wn1, bn1 = p['node1']
        wn2, bn2 = p['node2']

        wxp_a = _expand_rel(w1[:, 2 * H:].T)
        wxp_b = _expand_rel(we1[:, 2 * H:].T)
        w2b = _blockdiag(jnp.pad(w2.T, ((0, 0), (0, 13))), 8)
        e2b = _blockdiag(we2.T, 2)
        b2p = jnp.tile(be2, 2).reshape(1, 128)
        ones16 = jnp.ones((16,), f32)
        mones = jnp.einsum('qp,t,c->qtpc', jnp.eye(8, dtype=f32),
                           ones16, ones16).reshape(128, 128)
        wz = jnp.einsum('qp,t,c->qtpc', jnp.eye(8, dtype=f32),
                        jnp.pad(we[0], (0, 13)), ones16).reshape(128, 128)
        msc = jnp.stack([p['scale'][0], be[0]]).reshape(1, 2)
        msc = jnp.pad(msc, ((0, 0), (0, 6)))

        xg, relp = _sc_gather4(h, coords, row, col)
        delta = _edge_a_call(xg, relp,
                             w1[:, :2 * H].T, wxp_a,
                             _r(b1, 2 * H), w2b, mones, wz, msc)
        q = _sc_scatter16(delta.reshape(E, 16), row, zeros16)
        coords = _cupd_call(coords, q[0], q[1])
        relp2 = _sc_gather2(coords, row, col)
        e = _edge_b_call(xg, relp2,
                         we1[:, :2 * H].T, wxp_b,
                         _r(be1, 2 * H), e2b, b2p)
        ag = _sc_scatter64(e.reshape(E, H), row, zeros64)
        h = _node_call(h, ag[0], ag[1],
                       wn1[:, :H].T, wn1[:, H:].T, _r(bn1, 2 * H),
                       wn2.T, _r(bn2, H))

    wg1, bg1 = params['gate1']
    wg2, bg2 = params['gate2']
    wg3, bg3 = params['gate3']
    wo, bo = params['output_proj']
    g3t = jnp.pad(wg3.T, ((0, 0), (0, 7)))
    g3b = jnp.pad(bg3.reshape(1, 1), ((0, 0), (0, 7)))
    return _head_call(h, bcol, wg1.T, _r(bg1, H),
                      _r(params['ln_g'], H), _r(params['ln_b'], H),
                      wg2.T, _r(bg2, H // 2), g3t, g3b,
                      wo.T, _r(bo, OUT))


# speedup vs baseline: 4.5937x; 1.1510x over previous
import functools

import jax
import jax.numpy as jnp
from jax import lax
from jax.experimental import pallas as pl
from jax.experimental.pallas import tpu as pltpu
from jax.experimental.pallas import tpu_sc as plsc

N = 10000
E = 320000
D_IN = 128
H = 64
OUT = 128
NPAD = 10240

NC = 2
NS = 16
NW = NC * NS
EPW = E // NW
K = 80
NIT = EPW // K
RPT = NPAD // NS

_mesh = plsc.VectorSubcoreMesh(core_axis_name="c", subcore_axis_name="s")
_sc_params = pltpu.CompilerParams(use_tc_tiling_on_sc=False)

f32 = jnp.float32
i32 = jnp.int32


@functools.partial(
    pl.kernel,
    out_type=(
        jax.ShapeDtypeStruct((E, 2 * H), f32),
        jax.ShapeDtypeStruct((E // 8, 128), f32),
    ),
    mesh=_mesh,
    compiler_params=_sc_params,
    scratch_types=[
        pltpu.VMEM((K,), i32),
        pltpu.VMEM((K,), i32),
        pltpu.VMEM((K,), i32),
        pltpu.VMEM((K,), i32),
        pltpu.VMEM((K, H), f32),
        pltpu.VMEM((K, H), f32),
        pltpu.VMEM((K, H), f32),
        pltpu.VMEM((K, H), f32),
        pltpu.VMEM((K, 16), f32),
        pltpu.VMEM((K, 16), f32),
        pltpu.VMEM((K, 16), f32),
        pltpu.VMEM((K, 16), f32),
        pltpu.VMEM((K // 8, 128), f32),
        pltpu.SemaphoreType.DMA,
        pltpu.SemaphoreType.DMA,
    ],
)
def _sc_gather4(h_hbm, c_hbm, row_hbm, col_hbm,
                xg_hbm, rel_hbm,
                ir0, ic0, ir1, ic1, bxr0, bxc0, bxr1, bxc1,
                bcr0, bcc0, bcr1, bcc1, brel, sem_g, sem_w):
    wid = lax.axis_index("s") * NC + lax.axis_index("c")
    base = wid * EPW
    L = 16

    def pack(bcr, bcc):
        def sub(r, carry2):
            brel[r // 8, pl.ds(L * (r % 8), L)] = bcr[r, :] - bcc[r, :]
            return carry2

        lax.fori_loop(0, K, sub, 0)

    def step(j, ir, ic, bxr, bxc, bcr, bcc,
             pir, pic, pxr, pxc, pcr, pcc):
        off = base + j * K
        g1 = pltpu.async_copy(h_hbm.at[ir], bxr, sem_g)
        g2 = pltpu.async_copy(h_hbm.at[ic], bxc, sem_g)
        g3 = pltpu.async_copy(c_hbm.at[ir], bcr, sem_g)
        g4 = pltpu.async_copy(c_hbm.at[ic], bcc, sem_g)

        @pl.when(j + 1 < NIT)
        def _():
            pltpu.sync_copy(row_hbm.at[pl.ds(off + K, K)], pir)
            pltpu.sync_copy(col_hbm.at[pl.ds(off + K, K)], pic)

        @pl.when(j > 0)
        def _():
            pack(pcr, pcc)
            pltpu.async_copy(pxr, xg_hbm.at[pl.ds(off - K, K),
                                            pl.ds(0, H)], sem_w)
            pltpu.async_copy(pxc, xg_hbm.at[pl.ds(off - K, K),
                                            pl.ds(H, H)], sem_w)
            pltpu.async_copy(brel, rel_hbm.at[pl.ds((off - K) // 8,
                                              K // 8)], sem_w)

        g1.wait(); g2.wait(); g3.wait(); g4.wait()

        @pl.when(j > 0)
        def _():
            dw1 = pltpu.make_async_copy(pxr, xg_hbm.at[pl.ds(off - K, K),
                                                       pl.ds(0, H)], sem_w)
            dw2 = pltpu.make_async_copy(pxc, xg_hbm.at[pl.ds(off - K, K),
                                                       pl.ds(H, H)], sem_w)
            dw3 = pltpu.make_async_copy(brel, rel_hbm.at[pl.ds((off - K) // 8,
                                                         K // 8)], sem_w)
            dw1.wait(); dw2.wait(); dw3.wait()

    pltpu.sync_copy(row_hbm.at[pl.ds(base, K)], ir0)
    pltpu.sync_copy(col_hbm.at[pl.ds(base, K)], ic0)

    def body(j, carry):
        @pl.when(j % 2 == 0)
        def _():
            step(j, ir0, ic0, bxr0, bxc0, bcr0, bcc0,
                 ir1, ic1, bxr1, bxc1, bcr1, bcc1)

        @pl.when(j % 2 == 1)
        def _():
            step(j, ir1, ic1, bxr1, bxc1, bcr1, bcc1,
                 ir0, ic0, bxr0, bxc0, bcr0, bcc0)

        return carry

    lax.fori_loop(0, NIT, body, 0)
    offl = base + (NIT - 1) * K
    pack(bcr0, bcc0)
    pltpu.sync_copy(bxr0, xg_hbm.at[pl.ds(offl, K), pl.ds(0, H)])
    pltpu.sync_copy(bxc0, xg_hbm.at[pl.ds(offl, K), pl.ds(H, H)])
    pltpu.sync_copy(brel, rel_hbm.at[pl.ds(offl // 8, K // 8)])


@functools.partial(
    pl.kernel,
    out_type=jax.ShapeDtypeStruct((E // 8, 128), f32),
    mesh=_mesh,
    compiler_params=_sc_params,
    scratch_types=[
        pltpu.VMEM((K,), i32),
        pltpu.VMEM((K,), i32),
        pltpu.VMEM((K,), i32),
        pltpu.VMEM((K,), i32),
        pltpu.VMEM((K, 16), f32),
        pltpu.VMEM((K, 16), f32),
        pltpu.VMEM((K, 16), f32),
        pltpu.VMEM((K, 16), f32),
        pltpu.VMEM((K // 8, 128), f32),
        pltpu.SemaphoreType.DMA,
        pltpu.SemaphoreType.DMA,
    ],
)
def _sc_gather2(c_hbm, row_hbm, col_hbm, rel_hbm,
                ir0, ic0, ir1, ic1, bcr0, bcc0, bcr1, bcc1,
                brel, sem_g, sem_w):
    wid = lax.axis_index("s") * NC + lax.axis_index("c")
    base = wid * EPW
    L = 16

    def pack(bcr, bcc):
        def sub(r, carry2):
            brel[r // 8, pl.ds(L * (r % 8), L)] = bcr[r, :] - bcc[r, :]
            return carry2

        lax.fori_loop(0, K, sub, 0)

    def step(j, ir, ic, bcr, bcc, pir, pic, pcr, pcc):
        off = base + j * K
        g1 = pltpu.async_copy(c_hbm.at[ir], bcr, sem_g)
        g2 = pltpu.async_copy(c_hbm.at[ic], bcc, sem_g)

        @pl.when(j + 1 < NIT)
        def _():
            pltpu.sync_copy(row_hbm.at[pl.ds(off + K, K)], pir)
            pltpu.sync_copy(col_hbm.at[pl.ds(off + K, K)], pic)

        @pl.when(j > 0)
        def _():
            pack(pcr, pcc)
            pltpu.async_copy(brel, rel_hbm.at[pl.ds((off - K) // 8,
                                              K // 8)], sem_w)

        g1.wait(); g2.wait()

        @pl.when(j > 0)
        def _():
            pltpu.make_async_copy(brel, rel_hbm.at[pl.ds((off - K) // 8,
                                                   K // 8)], sem_w).wait()

    pltpu.sync_copy(row_hbm.at[pl.ds(base, K)], ir0)
    pltpu.sync_copy(col_hbm.at[pl.ds(base, K)], ic0)

    def body(j, carry):
        @pl.when(j % 2 == 0)
        def _():
            step(j, ir0, ic0, bcr0, bcc0, ir1, ic1, bcr1, bcc1)

        @pl.when(j % 2 == 1)
        def _():
            step(j, ir1, ic1, bcr1, bcc1, ir0, ic0, bcr0, bcc0)

        return carry

    lax.fori_loop(0, NIT, body, 0)
    offl = base + (NIT - 1) * K
    pack(bcr0, bcc0)
    pltpu.sync_copy(brel, rel_hbm.at[pl.ds(offl // 8, K // 8)])


def _make_sc_scatter(W):
    @functools.partial(
        pl.kernel,
        out_type=jax.ShapeDtypeStruct((2, NPAD, W), f32),
        mesh=_mesh,
        compiler_params=_sc_params,
        scratch_types=[
            pltpu.VMEM((K,), i32),
            pltpu.VMEM((K,), i32),
            pltpu.VMEM((K, W), f32),
            pltpu.VMEM((K, W), f32),
            pltpu.VMEM_SHARED((NPAD, W), f32),
            pltpu.SemaphoreType.DMA,
            pltpu.SemaphoreType.DMA,
        ],
    )
    def scat(val_hbm, row_hbm, zero_hbm, out_hbm,
             i0, i1, v0, v1, acc, sem_s, sem_v):
        cid = lax.axis_index("c")
        sid = lax.axis_index("s")
        wid = sid * NC + cid
        pltpu.sync_copy(zero_hbm.at[pl.ds(sid * RPT, RPT)],
                        acc.at[pl.ds(sid * RPT, RPT)])
        plsc.subcore_barrier()
        base = wid * EPW

        def step(j, ci, cv, pi, pv):
            s = pltpu.async_copy(cv, acc.at[ci], sem_s, add=True)

            @pl.when(j + 1 < NIT)
            def _():
                off2 = base + (j + 1) * K
                l1 = pltpu.async_copy(row_hbm.at[pl.ds(off2, K)], pi, sem_v)
                l2 = pltpu.async_copy(val_hbm.at[pl.ds(off2, K)], pv, sem_v)
                l1.wait(); l2.wait()

            s.wait()

        pltpu.sync_copy(row_hbm.at[pl.ds(base, K)], i0)
        pltpu.sync_copy(val_hbm.at[pl.ds(base, K)], v0)

        def body(j, carry):
            @pl.when(j % 2 == 0)
            def _():
                step(j, i0, v0, i1, v1)

            @pl.when(j % 2 == 1)
            def _():
                step(j, i1, v1, i0, v0)

            return carry

        lax.fori_loop(0, NIT, body, 0)
        plsc.subcore_barrier()
        pltpu.sync_copy(acc.at[pl.ds(sid * RPT, RPT)],
                        out_hbm.at[cid, pl.ds(sid * RPT, RPT)])

    return scat


_sc_scatter16 = _make_sc_scatter(16)
_sc_scatter64 = _make_sc_scatter(H)


EB = 2560
NBK = 2048


def _full(shape):
    return pl.BlockSpec(shape, lambda *_: tuple(0 for _ in shape))


def _rows(bs, w):
    return pl.BlockSpec((bs, w), lambda i, *_: (i, 0))


def _proj_body(x_ref, wt, b, o_ref):
    o_ref[...] = (jnp.dot(x_ref[...], wt[...], preferred_element_type=f32)
                  + b[...])


def _edge_a_body(xg, relp, w1x, wxp, b1, w2b, mones, wz, msc, dout):
    urel = jnp.dot(relp[...], wxp[...],
                   preferred_element_type=f32).reshape(EB, 2 * H)
    u = jnp.dot(xg[...], w1x[...], preferred_element_type=f32) + urel + b1[...]
    u = u * jax.nn.sigmoid(u)
    up = u.reshape(EB // 8, 8 * 2 * H)
    dp = jnp.dot(up, w2b[...], preferred_element_type=f32)
    n2 = jnp.dot(dp * dp, mones[...], preferred_element_type=f32)
    nrm = jnp.maximum(jnp.sqrt(n2), 1e-8)
    ew = jax.nn.sigmoid(jnp.dot(relp[...], wz[...], preferred_element_type=f32)
                        + msc[0, 1])
    dout[...] = dp * ((msc[0, 0] * ew) / nrm)


def _edge_b_body(xg, relp, e1x, wxp, b1, e2b, b2p, eout):
    urel = jnp.dot(relp[...], wxp[...],
                   preferred_element_type=f32).reshape(EB, 2 * H)
    u = jnp.dot(xg[...], e1x[...], preferred_element_type=f32) + urel + b1[...]
    u = u * jax.nn.sigmoid(u)
    up = u.reshape(EB // 2, 2 * 2 * H)
    v = jnp.dot(up, e2b[...], preferred_element_type=f32) + b2p[...]
    eout[...] = v * jax.nn.sigmoid(v)


def _cupd_body(c, q0, q1, cout):
    cout[...] = c[...] + q0[...] + q1[...]


def _node_body(h, p0, p1, n1h, n1a, b1, n2t, b2, hout):
    agg = p0[...] + p1[...]
    u = (jnp.dot(h[...], n1h[...], preferred_element_type=f32)
         + jnp.dot(agg, n1a[...], preferred_element_type=f32) + b1[...])
    u = u * jax.nn.sigmoid(u)
    hout[...] = h[...] + jnp.dot(u, n2t[...], preferred_element_type=f32) + b2[...]


def _head_body(h, bcol, g1t, g1b, lng, lnb, g2t, g2b, g3t, g3b, wot, bo,
               out, m_ref, d_ref, p_ref):
    ph = pl.program_id(0)
    i = pl.program_id(1)
    G = pl.num_programs(1)

    @pl.when(jnp.logical_and(ph == 0, i == 0))
    def _():
        m_ref[...] = jnp.full((1, 8), -1e30, f32)

    @pl.when(jnp.logical_and(ph == 1, i == 0))
    def _():
        d_ref[...] = jnp.zeros((8, 8), f32)
        p_ref[...] = jnp.zeros((8, H), f32)

    g = jnp.dot(h[...], g1t[...], preferred_element_type=f32) + g1b[...]
    mu = jnp.mean(g, axis=-1, keepdims=True)
    var = jnp.mean((g - mu) ** 2, axis=-1, keepdims=True)
    g = (g - mu) * lax.rsqrt(var + 1e-5) * lng[...] + lnb[...]
    g = jnp.maximum(g, 0.0)
    q = jnp.maximum(jnp.dot(g, g2t[...], preferred_element_type=f32)
                    + g2b[...], 0.0)
    s = jnp.dot(q, g3t[...], preferred_element_type=f32) + g3b[...]
    s0 = s[:, 0:1]
    oh = bcol[...] == lax.broadcasted_iota(i32, (1, 8), 1)

    @pl.when(ph == 0)
    def _():
        sm = jnp.where(oh, s0, -1e30)
        bm = jnp.max(sm, axis=0, keepdims=True)
        m_ref[...] = jnp.maximum(m_ref[...], bm)

    @pl.when(ph == 1)
    def _():
        w = jnp.where(oh, jnp.exp(s0 - m_ref[...]), 0.0)
        dn = (((0,), (0,)), ((), ()))
        d_ref[...] += lax.dot_general(w, jnp.ones((NBK, 8), f32), dn,
                                      preferred_element_type=f32)
        p_ref[...] += lax.dot_general(w, h[...], dn,
                                      preferred_element_type=f32)

    @pl.when(jnp.logical_and(ph == 1, i == G - 1))
    def _():
        pooled = p_ref[...] / d_ref[:, 0:1]
        out[...] = (jnp.dot(pooled, wot[...], preferred_element_type=f32)
                    + bo[...])


def _proj_call(xp, wt, b):
    return pl.pallas_call(
        _proj_body,
        grid=(NPAD // NBK,),
        in_specs=[_rows(NBK, D_IN), _full((D_IN, H)), _full((1, H))],
        out_specs=_rows(NBK, H),
        out_shape=jax.ShapeDtypeStruct((NPAD, H), f32),
    )(xp, wt, b)


def _edge_a_call(xg, relp, w1x, wxp, b1, w2b, mones, wz, msc):
    return pl.pallas_call(
        _edge_a_body,
        grid=(E // EB,),
        in_specs=[_rows(EB, 2 * H), _rows(EB // 8, 128),
                  _full((2 * H, 2 * H)), _full((128, 8 * 2 * H)),
                  _full((1, 2 * H)), _full((8 * 2 * H, 128)),
                  _full((128, 128)), _full((128, 128)),
                  pl.BlockSpec(memory_space=pltpu.SMEM)],
        out_specs=_rows(EB // 8, 128),
        out_shape=jax.ShapeDtypeStruct((E // 8, 128), f32),
    )(xg, relp, w1x, wxp, b1, w2b, mones, wz, msc)


def _edge_b_call(xg, relp, e1x, wxp, b1, e2b, b2p):
    return pl.pallas_call(
        _edge_b_body,
        grid=(E // EB,),
        in_specs=[_rows(EB, 2 * H), _rows(EB // 8, 128),
                  _full((2 * H, 2 * H)), _full((128, 8 * 2 * H)),
                  _full((1, 2 * H)), _full((2 * 2 * H, 128)), _full((1, 128))],
        out_specs=_rows(EB // 2, 128),
        out_shape=jax.ShapeDtypeStruct((E // 2, 128), f32),
    )(xg, relp, e1x, wxp, b1, e2b, b2p)


def _cupd_call(c, q0, q1):
    return pl.pallas_call(
        _cupd_body,
        grid=(NPAD // NBK,),
        in_specs=[_rows(NBK, 16)] * 3,
        out_specs=_rows(NBK, 16),
        out_shape=jax.ShapeDtypeStruct((NPAD, 16), f32),
    )(c, q0, q1)


def _node_call(h, p0, p1, n1h, n1a, b1, n2t, b2):
    return pl.pallas_call(
        _node_body,
        grid=(NPAD // NBK,),
        in_specs=[_rows(NBK, H), _rows(NBK, H), _rows(NBK, H),
                  _full((H, 2 * H)), _full((H, 2 * H)), _full((1, 2 * H)),
                  _full((2 * H, H)), _full((1, H))],
        out_specs=_rows(NBK, H),
        out_shape=jax.ShapeDtypeStruct((NPAD, H), f32),
    )(h, p0, p1, n1h, n1a, b1, n2t, b2)


def _head_call(h, bcol, g1t, g1b, lng, lnb, g2t, g2b, g3t, g3b, wot, bo):
    return pl.pallas_call(
        _head_body,
        grid=(2, NPAD // NBK),
        in_specs=[pl.BlockSpec((NBK, H), lambda p, i: (i, 0)),
                  pl.BlockSpec((NBK, 1), lambda p, i: (i, 0)),
                  _full((H, H)), _full((1, H)), _full((1, H)), _full((1, H)),
                  _full((H, H // 2)), _full((1, H // 2)),
                  _full((H // 2, 8)), _full((1, 8)),
                  _full((H, OUT)), _full((1, OUT))],
        out_specs=pl.BlockSpec((8, OUT), lambda p, i: (0, 0)),
        out_shape=jax.ShapeDtypeStruct((8, OUT), f32),
        scratch_shapes=[pltpu.VMEM((1, 8), f32), pltpu.VMEM((8, 8), f32),
                        pltpu.VMEM((8, H), f32)],
    )(h, bcol, g1t, g1b, lng, lnb, g2t, g2b, g3t, g3b, wot, bo)


def _r(v, n):
    return v.reshape(1, n)


def _expand_rel(w3):
    wp = jnp.pad(w3, ((0, 13), (0, 0)))
    return jnp.einsum('qp,tc->qtpc', jnp.eye(8, dtype=f32),
                      wp).reshape(128, 8 * 128)


def _blockdiag(wt, k):
    a, b = wt.shape
    return jnp.einsum('pq,dc->pdqc', jnp.eye(k, dtype=f32),
                      wt).reshape(k * a, k * b)


def kernel(x, pos, edge_index, batch, params):
    row = edge_index[0].astype(i32)
    col = edge_index[1].astype(i32)

    xp = jnp.pad(x, ((0, NPAD - N), (0, 0)))
    coords = jnp.pad(pos, ((0, NPAD - N), (0, 13)))
    bcol = jnp.pad(batch.astype(i32), (0, NPAD - N),
                   constant_values=127).reshape(NPAD, 1)
    zeros16 = jnp.zeros((NPAD, 16), f32)
    zeros64 = jnp.zeros((NPAD, H), f32)

    wi, bi = params['input_proj']
    h = _proj_call(xp, wi.T, _r(bi, H))

    for p in params['layers']:
        w1, b1 = p['coord1']
        w2, b2 = p['coord2']
        we, be = p['ew']
        we1, be1 = p['edge1']
        we2, be2 = p['edge2']
        wn1, bn1 = p['node1']
        wn2, bn2 = p['node2']

        wxp_a = _expand_rel(w1[:, 2 * H:].T)
        wxp_b = _expand_rel(we1[:, 2 * H:].T)
        w2b = _blockdiag(jnp.pad(w2.T, ((0, 0), (0, 13))), 8)
        e2b = _blockdiag(we2.T, 2)
        b2p = jnp.tile(be2, 2).reshape(1, 128)
        ones16 = jnp.ones((16,), f32)
        mones = jnp.einsum('qp,t,c->qtpc', jnp.eye(8, dtype=f32),
                           ones16, ones16).reshape(128, 128)
        wz = jnp.einsum('qp,t,c->qtpc', jnp.eye(8, dtype=f32),
                        jnp.pad(we[0], (0, 13)), ones16).reshape(128, 128)
        msc = jnp.stack([p['scale'][0], be[0]]).reshape(1, 2)
        msc = jnp.pad(msc, ((0, 0), (0, 6)))

        xg, relp = _sc_gather4(h, coords, row, col)
        delta = _edge_a_call(xg, relp,
                             w1[:, :2 * H].T, wxp_a,
                             _r(b1, 2 * H), w2b, mones, wz, msc)
        q = _sc_scatter16(delta.reshape(E, 16), row, zeros16)
        coords = _cupd_call(coords, q[0], q[1])
        relp2 = _sc_gather2(coords, row, col)
        e = _edge_b_call(xg, relp2,
                         we1[:, :2 * H].T, wxp_b,
                         _r(be1, 2 * H), e2b, b2p)
        ag = _sc_scatter64(e.reshape(E, H), row, zeros64)
        h = _node_call(h, ag[0], ag[1],
                       wn1[:, :H].T, wn1[:, H:].T, _r(bn1, 2 * H),
                       wn2.T, _r(bn2, H))

    wg1, bg1 = params['gate1']
    wg2, bg2 = params['gate2']
    wg3, bg3 = params['gate3']
    wo, bo = params['output_proj']
    g3t = jnp.pad(wg3.T, ((0, 0), (0, 7)))
    g3b = jnp.pad(bg3.reshape(1, 1), ((0, 0), (0, 7)))
    return _head_call(h, bcol, wg1.T, _r(bg1, H),
                      _r(params['ln_g'], H), _r(params['ln_b'], H),
                      wg2.T, _r(bg2, H // 2), g3t, g3b,
                      wo.T, _r(bo, OUT))


# speedup vs baseline: 4.9441x; 1.0763x over previous
import functools

import jax
import jax.numpy as jnp
from jax import lax
from jax.experimental import pallas as pl
from jax.experimental.pallas import tpu as pltpu
from jax.experimental.pallas import tpu_sc as plsc

N = 10000
E = 320000
D_IN = 128
H = 64
OUT = 128
NPAD = 10240

NC = 2
NS = 16
NW = NC * NS
EPW = E // NW
K = 80
NIT = EPW // K
RPT = NPAD // NS

_mesh = plsc.VectorSubcoreMesh(core_axis_name="c", subcore_axis_name="s")
_sc_params = pltpu.CompilerParams(use_tc_tiling_on_sc=False)

f32 = jnp.float32
i32 = jnp.int32


@functools.partial(
    pl.kernel,
    out_type=(
        jax.ShapeDtypeStruct((E, 2 * H), f32),
        jax.ShapeDtypeStruct((E // 8, 128), f32),
    ),
    mesh=_mesh,
    compiler_params=_sc_params,
    scratch_types=[
        pltpu.VMEM((K,), i32),
        pltpu.VMEM((K,), i32),
        pltpu.VMEM((K,), i32),
        pltpu.VMEM((K,), i32),
        pltpu.VMEM((K, H), f32),
        pltpu.VMEM((K, H), f32),
        pltpu.VMEM((K, H), f32),
        pltpu.VMEM((K, H), f32),
        pltpu.VMEM((K, 16), f32),
        pltpu.VMEM((K, 16), f32),
        pltpu.VMEM((K, 16), f32),
        pltpu.VMEM((K, 16), f32),
        pltpu.VMEM((K // 8, 128), f32),
        pltpu.SemaphoreType.DMA,
        pltpu.SemaphoreType.DMA,
    ],
)
def _sc_gather4(h_hbm, c_hbm, row_hbm, col_hbm,
                xg_hbm, rel_hbm,
                ir0, ic0, ir1, ic1, bxr0, bxc0, bxr1, bxc1,
                bcr0, bcc0, bcr1, bcc1, brel, sem_g, sem_w):
    wid = lax.axis_index("s") * NC + lax.axis_index("c")
    base = wid * EPW
    L = 16

    def pack(bcr, bcc):
        def sub(r, carry2):
            brel[r // 8, pl.ds(L * (r % 8), L)] = bcr[r, :] - bcc[r, :]
            return carry2

        lax.fori_loop(0, K, sub, 0)

    def step(j, ir, ic, bxr, bxc, bcr, bcc,
             pir, pic, pxr, pxc, pcr, pcc):
        off = base + j * K
        g1 = pltpu.async_copy(h_hbm.at[ir], bxr, sem_g)
        g2 = pltpu.async_copy(h_hbm.at[ic], bxc, sem_g)
        g3 = pltpu.async_copy(c_hbm.at[ir], bcr, sem_g)
        g4 = pltpu.async_copy(c_hbm.at[ic], bcc, sem_g)

        @pl.when(j + 1 < NIT)
        def _():
            pltpu.sync_copy(row_hbm.at[pl.ds(off + K, K)], pir)
            pltpu.sync_copy(col_hbm.at[pl.ds(off + K, K)], pic)

        @pl.when(j > 0)
        def _():
            pack(pcr, pcc)
            pltpu.async_copy(pxr, xg_hbm.at[pl.ds(off - K, K),
                                            pl.ds(0, H)], sem_w)
            pltpu.async_copy(pxc, xg_hbm.at[pl.ds(off - K, K),
                                            pl.ds(H, H)], sem_w)
            pltpu.async_copy(brel, rel_hbm.at[pl.ds((off - K) // 8,
                                              K // 8)], sem_w)

        g1.wait(); g2.wait(); g3.wait(); g4.wait()

        @pl.when(j > 0)
        def _():
            dw1 = pltpu.make_async_copy(pxr, xg_hbm.at[pl.ds(off - K, K),
                                                       pl.ds(0, H)], sem_w)
            dw2 = pltpu.make_async_copy(pxc, xg_hbm.at[pl.ds(off - K, K),
                                                       pl.ds(H, H)], sem_w)
            dw3 = pltpu.make_async_copy(brel, rel_hbm.at[pl.ds((off - K) // 8,
                                                         K // 8)], sem_w)
            dw1.wait(); dw2.wait(); dw3.wait()

    pltpu.sync_copy(row_hbm.at[pl.ds(base, K)], ir0)
    pltpu.sync_copy(col_hbm.at[pl.ds(base, K)], ic0)

    def body(j, carry):
        @pl.when(j % 2 == 0)
        def _():
            step(j, ir0, ic0, bxr0, bxc0, bcr0, bcc0,
                 ir1, ic1, bxr1, bxc1, bcr1, bcc1)

        @pl.when(j % 2 == 1)
        def _():
            step(j, ir1, ic1, bxr1, bxc1, bcr1, bcc1,
                 ir0, ic0, bxr0, bxc0, bcr0, bcc0)

        return carry

    lax.fori_loop(0, NIT, body, 0)
    offl = base + (NIT - 1) * K
    pack(bcr0, bcc0)
    pltpu.sync_copy(bxr0, xg_hbm.at[pl.ds(offl, K), pl.ds(0, H)])
    pltpu.sync_copy(bxc0, xg_hbm.at[pl.ds(offl, K), pl.ds(H, H)])
    pltpu.sync_copy(brel, rel_hbm.at[pl.ds(offl // 8, K // 8)])


@functools.partial(
    pl.kernel,
    out_type=jax.ShapeDtypeStruct((E // 8, 128), f32),
    mesh=_mesh,
    compiler_params=_sc_params,
    scratch_types=[
        pltpu.VMEM((K,), i32),
        pltpu.VMEM((K,), i32),
        pltpu.VMEM((K,), i32),
        pltpu.VMEM((K,), i32),
        pltpu.VMEM((K, 16), f32),
        pltpu.VMEM((K, 16), f32),
        pltpu.VMEM((K, 16), f32),
        pltpu.VMEM((K, 16), f32),
        pltpu.VMEM((K // 8, 128), f32),
        pltpu.SemaphoreType.DMA,
        pltpu.SemaphoreType.DMA,
    ],
)
def _sc_gather2(c_hbm, row_hbm, col_hbm, rel_hbm,
                ir0, ic0, ir1, ic1, bcr0, bcc0, bcr1, bcc1,
                brel, sem_g, sem_w):
    wid = lax.axis_index("s") * NC + lax.axis_index("c")
    base = wid * EPW
    L = 16

    def pack(bcr, bcc):
        def sub(r, carry2):
            brel[r // 8, pl.ds(L * (r % 8), L)] = bcr[r, :] - bcc[r, :]
            return carry2

        lax.fori_loop(0, K, sub, 0)

    def step(j, ir, ic, bcr, bcc, pir, pic, pcr, pcc):
        off = base + j * K
        g1 = pltpu.async_copy(c_hbm.at[ir], bcr, sem_g)
        g2 = pltpu.async_copy(c_hbm.at[ic], bcc, sem_g)

        @pl.when(j + 1 < NIT)
        def _():
            pltpu.sync_copy(row_hbm.at[pl.ds(off + K, K)], pir)
            pltpu.sync_copy(col_hbm.at[pl.ds(off + K, K)], pic)

        @pl.when(j > 0)
        def _():
            pack(pcr, pcc)
            pltpu.async_copy(brel, rel_hbm.at[pl.ds((off - K) // 8,
                                              K // 8)], sem_w)

        g1.wait(); g2.wait()

        @pl.when(j > 0)
        def _():
            pltpu.make_async_copy(brel, rel_hbm.at[pl.ds((off - K) // 8,
                                                   K // 8)], sem_w).wait()

    pltpu.sync_copy(row_hbm.at[pl.ds(base, K)], ir0)
    pltpu.sync_copy(col_hbm.at[pl.ds(base, K)], ic0)

    def body(j, carry):
        @pl.when(j % 2 == 0)
        def _():
            step(j, ir0, ic0, bcr0, bcc0, ir1, ic1, bcr1, bcc1)

        @pl.when(j % 2 == 1)
        def _():
            step(j, ir1, ic1, bcr1, bcc1, ir0, ic0, bcr0, bcc0)

        return carry

    lax.fori_loop(0, NIT, body, 0)
    offl = base + (NIT - 1) * K
    pack(bcr0, bcc0)
    pltpu.sync_copy(brel, rel_hbm.at[pl.ds(offl // 8, K // 8)])


def _make_sc_scatter(W):
    @functools.partial(
        pl.kernel,
        out_type=jax.ShapeDtypeStruct((2, NPAD, W), f32),
        mesh=_mesh,
        compiler_params=_sc_params,
        scratch_types=[
            pltpu.VMEM((K,), i32),
            pltpu.VMEM((K,), i32),
            pltpu.VMEM((K, W), f32),
            pltpu.VMEM((K, W), f32),
            pltpu.VMEM_SHARED((NPAD, W), f32),
            pltpu.SemaphoreType.DMA,
            pltpu.SemaphoreType.DMA,
        ],
    )
    def scat(val_hbm, row_hbm, zero_hbm, out_hbm,
             i0, i1, v0, v1, acc, sem_s, sem_v):
        cid = lax.axis_index("c")
        sid = lax.axis_index("s")
        wid = sid * NC + cid
        pltpu.sync_copy(zero_hbm.at[pl.ds(sid * RPT, RPT)],
                        acc.at[pl.ds(sid * RPT, RPT)])
        plsc.subcore_barrier()
        base = wid * EPW

        def step(j, ci, cv, pi, pv):
            s = pltpu.async_copy(cv, acc.at[ci], sem_s, add=True)

            @pl.when(j + 1 < NIT)
            def _():
                off2 = base + (j + 1) * K
                l1 = pltpu.async_copy(row_hbm.at[pl.ds(off2, K)], pi, sem_v)
                l2 = pltpu.async_copy(val_hbm.at[pl.ds(off2, K)], pv, sem_v)
                l1.wait(); l2.wait()

            s.wait()

        pltpu.sync_copy(row_hbm.at[pl.ds(base, K)], i0)
        pltpu.sync_copy(val_hbm.at[pl.ds(base, K)], v0)

        def body(j, carry):
            @pl.when(j % 2 == 0)
            def _():
                step(j, i0, v0, i1, v1)

            @pl.when(j % 2 == 1)
            def _():
                step(j, i1, v1, i0, v0)

            return carry

        lax.fori_loop(0, NIT, body, 0)
        plsc.subcore_barrier()
        pltpu.sync_copy(acc.at[pl.ds(sid * RPT, RPT)],
                        out_hbm.at[cid, pl.ds(sid * RPT, RPT)])

    return scat


_sc_scatter16 = _make_sc_scatter(16)
_sc_scatter64 = _make_sc_scatter(H)


EB = 6400
NBK = 2048


def _full(shape):
    return pl.BlockSpec(shape, lambda *_: tuple(0 for _ in shape))


def _rows(bs, w):
    return pl.BlockSpec((bs, w), lambda i, *_: (i, 0))


def _proj_body(x_ref, wt, b, o_ref):
    o_ref[...] = (jnp.dot(x_ref[...], wt[...], preferred_element_type=f32)
                  + b[...])


def _edge_a_body(xg, relp, w1x, wxp, b1, w2b, mones, wz, msc, dout):
    urel = jnp.dot(relp[...], wxp[...],
                   preferred_element_type=f32).reshape(EB, 2 * H)
    u = jnp.dot(xg[...], w1x[...], preferred_element_type=f32) + urel + b1[...]
    u = u * jax.nn.sigmoid(u)
    up = u.reshape(EB // 8, 8 * 2 * H)
    dp = jnp.dot(up, w2b[...], preferred_element_type=f32)
    n2 = jnp.dot(dp * dp, mones[...], preferred_element_type=f32)
    nrm = jnp.maximum(jnp.sqrt(n2), 1e-8)
    ew = jax.nn.sigmoid(jnp.dot(relp[...], wz[...], preferred_element_type=f32)
                        + msc[0, 1])
    dout[...] = dp * ((msc[0, 0] * ew) / nrm)


def _edge_b_body(xg, relp, e1x, wxp, b1, e2b, b2p, eout):
    urel = jnp.dot(relp[...], wxp[...],
                   preferred_element_type=f32).reshape(EB, 2 * H)
    u = jnp.dot(xg[...], e1x[...], preferred_element_type=f32) + urel + b1[...]
    u = u * jax.nn.sigmoid(u)
    up = u.reshape(EB // 2, 2 * 2 * H)
    v = jnp.dot(up, e2b[...], preferred_element_type=f32) + b2p[...]
    eout[...] = v * jax.nn.sigmoid(v)


def _cupd_body(c, q0, q1, cout):
    cout[...] = c[...] + q0[...] + q1[...]


def _node_body(h, p0, p1, n1h, n1a, b1, n2t, b2, hout):
    agg = p0[...] + p1[...]
    u = (jnp.dot(h[...], n1h[...], preferred_element_type=f32)
         + jnp.dot(agg, n1a[...], preferred_element_type=f32) + b1[...])
    u = u * jax.nn.sigmoid(u)
    hout[...] = h[...] + jnp.dot(u, n2t[...], preferred_element_type=f32) + b2[...]


def _head_body(h, bcol, g1t, g1b, lng, lnb, g2t, g2b, g3t, g3b, wot, bo,
               out, m_ref, d_ref, p_ref):
    ph = pl.program_id(0)
    i = pl.program_id(1)
    G = pl.num_programs(1)

    @pl.when(jnp.logical_and(ph == 0, i == 0))
    def _():
        m_ref[...] = jnp.full((1, 8), -1e30, f32)

    @pl.when(jnp.logical_and(ph == 1, i == 0))
    def _():
        d_ref[...] = jnp.zeros((8, 8), f32)
        p_ref[...] = jnp.zeros((8, H), f32)

    g = jnp.dot(h[...], g1t[...], preferred_element_type=f32) + g1b[...]
    mu = jnp.mean(g, axis=-1, keepdims=True)
    var = jnp.mean((g - mu) ** 2, axis=-1, keepdims=True)
    g = (g - mu) * lax.rsqrt(var + 1e-5) * lng[...] + lnb[...]
    g = jnp.maximum(g, 0.0)
    q = jnp.maximum(jnp.dot(g, g2t[...], preferred_element_type=f32)
                    + g2b[...], 0.0)
    s = jnp.dot(q, g3t[...], preferred_element_type=f32) + g3b[...]
    s0 = s[:, 0:1]
    oh = bcol[...] == lax.broadcasted_iota(i32, (1, 8), 1)

    @pl.when(ph == 0)
    def _():
        sm = jnp.where(oh, s0, -1e30)
        bm = jnp.max(sm, axis=0, keepdims=True)
        m_ref[...] = jnp.maximum(m_ref[...], bm)

    @pl.when(ph == 1)
    def _():
        w = jnp.where(oh, jnp.exp(s0 - m_ref[...]), 0.0)
        dn = (((0,), (0,)), ((), ()))
        d_ref[...] += lax.dot_general(w, jnp.ones((NBK, 8), f32), dn,
                                      preferred_element_type=f32)
        p_ref[...] += lax.dot_general(w, h[...], dn,
                                      preferred_element_type=f32)

    @pl.when(jnp.logical_and(ph == 1, i == G - 1))
    def _():
        pooled = p_ref[...] / d_ref[:, 0:1]
        out[...] = (jnp.dot(pooled, wot[...], preferred_element_type=f32)
                    + bo[...])


def _proj_call(xp, wt, b):
    return pl.pallas_call(
        _proj_body,
        grid=(NPAD // NBK,),
        in_specs=[_rows(NBK, D_IN), _full((D_IN, H)), _full((1, H))],
        out_specs=_rows(NBK, H),
        out_shape=jax.ShapeDtypeStruct((NPAD, H), f32),
    )(xp, wt, b)


def _edge_a_call(xg, relp, w1x, wxp, b1, w2b, mones, wz, msc):
    return pl.pallas_call(
        _edge_a_body,
        grid=(E // EB,),
        in_specs=[_rows(EB, 2 * H), _rows(EB // 8, 128),
                  _full((2 * H, 2 * H)), _full((128, 8 * 2 * H)),
                  _full((1, 2 * H)), _full((8 * 2 * H, 128)),
                  _full((128, 128)), _full((128, 128)),
                  pl.BlockSpec(memory_space=pltpu.SMEM)],
        out_specs=_rows(EB // 8, 128),
        out_shape=jax.ShapeDtypeStruct((E // 8, 128), f32),
    )(xg, relp, w1x, wxp, b1, w2b, mones, wz, msc)


def _edge_b_call(xg, relp, e1x, wxp, b1, e2b, b2p):
    return pl.pallas_call(
        _edge_b_body,
        grid=(E // EB,),
        in_specs=[_rows(EB, 2 * H), _rows(EB // 8, 128),
                  _full((2 * H, 2 * H)), _full((128, 8 * 2 * H)),
                  _full((1, 2 * H)), _full((2 * 2 * H, 128)), _full((1, 128))],
        out_specs=_rows(EB // 2, 128),
        out_shape=jax.ShapeDtypeStruct((E // 2, 128), f32),
    )(xg, relp, e1x, wxp, b1, e2b, b2p)


def _cupd_call(c, q0, q1):
    return pl.pallas_call(
        _cupd_body,
        grid=(NPAD // NBK,),
        in_specs=[_rows(NBK, 16)] * 3,
        out_specs=_rows(NBK, 16),
        out_shape=jax.ShapeDtypeStruct((NPAD, 16), f32),
    )(c, q0, q1)


def _node_call(h, p0, p1, n1h, n1a, b1, n2t, b2):
    return pl.pallas_call(
        _node_body,
        grid=(NPAD // NBK,),
        in_specs=[_rows(NBK, H), _rows(NBK, H), _rows(NBK, H),
                  _full((H, 2 * H)), _full((H, 2 * H)), _full((1, 2 * H)),
                  _full((2 * H, H)), _full((1, H))],
        out_specs=_rows(NBK, H),
        out_shape=jax.ShapeDtypeStruct((NPAD, H), f32),
    )(h, p0, p1, n1h, n1a, b1, n2t, b2)


def _head_call(h, bcol, g1t, g1b, lng, lnb, g2t, g2b, g3t, g3b, wot, bo):
    return pl.pallas_call(
        _head_body,
        grid=(2, NPAD // NBK),
        in_specs=[pl.BlockSpec((NBK, H), lambda p, i: (i, 0)),
                  pl.BlockSpec((NBK, 1), lambda p, i: (i, 0)),
                  _full((H, H)), _full((1, H)), _full((1, H)), _full((1, H)),
                  _full((H, H // 2)), _full((1, H // 2)),
                  _full((H // 2, 8)), _full((1, 8)),
                  _full((H, OUT)), _full((1, OUT))],
        out_specs=pl.BlockSpec((8, OUT), lambda p, i: (0, 0)),
        out_shape=jax.ShapeDtypeStruct((8, OUT), f32),
        scratch_shapes=[pltpu.VMEM((1, 8), f32), pltpu.VMEM((8, 8), f32),
                        pltpu.VMEM((8, H), f32)],
    )(h, bcol, g1t, g1b, lng, lnb, g2t, g2b, g3t, g3b, wot, bo)


def _r(v, n):
    return v.reshape(1, n)


def _expand_rel(w3):
    wp = jnp.pad(w3, ((0, 13), (0, 0)))
    return jnp.einsum('qp,tc->qtpc', jnp.eye(8, dtype=f32),
                      wp).reshape(128, 8 * 128)


def _blockdiag(wt, k):
    a, b = wt.shape
    return jnp.einsum('pq,dc->pdqc', jnp.eye(k, dtype=f32),
                      wt).reshape(k * a, k * b)


def kernel(x, pos, edge_index, batch, params):
    row = edge_index[0].astype(i32)
    col = edge_index[1].astype(i32)

    xp = jnp.pad(x, ((0, NPAD - N), (0, 0)))
    coords = jnp.pad(pos, ((0, NPAD - N), (0, 13)))
    bcol = jnp.pad(batch.astype(i32), (0, NPAD - N),
                   constant_values=127).reshape(NPAD, 1)
    zeros16 = jnp.zeros((NPAD, 16), f32)
    zeros64 = jnp.zeros((NPAD, H), f32)

    wi, bi = params['input_proj']
    h = _proj_call(xp, wi.T, _r(bi, H))

    for p in params['layers']:
        w1, b1 = p['coord1']
        w2, b2 = p['coord2']
        we, be = p['ew']
        we1, be1 = p['edge1']
        we2, be2 = p['edge2']
        wn1, bn1 = p['node1']
        wn2, bn2 = p['node2']

        wxp_a = _expand_rel(w1[:, 2 * H:].T)
        wxp_b = _expand_rel(we1[:, 2 * H:].T)
        w2b = _blockdiag(jnp.pad(w2.T, ((0, 0), (0, 13))), 8)
        e2b = _blockdiag(we2.T, 2)
        b2p = jnp.tile(be2, 2).reshape(1, 128)
        ones16 = jnp.ones((16,), f32)
        mones = jnp.einsum('qp,t,c->qtpc', jnp.eye(8, dtype=f32),
                           ones16, ones16).reshape(128, 128)
        wz = jnp.einsum('qp,t,c->qtpc', jnp.eye(8, dtype=f32),
                        jnp.pad(we[0], (0, 13)), ones16).reshape(128, 128)
        msc = jnp.stack([p['scale'][0], be[0]]).reshape(1, 2)
        msc = jnp.pad(msc, ((0, 0), (0, 6)))

        xg, relp = _sc_gather4(h, coords, row, col)
        delta = _edge_a_call(xg, relp,
                             w1[:, :2 * H].T, wxp_a,
                             _r(b1, 2 * H), w2b, mones, wz, msc)
        q = _sc_scatter16(delta.reshape(E, 16), row, zeros16)
        coords = _cupd_call(coords, q[0], q[1])
        relp2 = _sc_gather2(coords, row, col)
        e = _edge_b_call(xg, relp2,
                         we1[:, :2 * H].T, wxp_b,
                         _r(be1, 2 * H), e2b, b2p)
        ag = _sc_scatter64(e.reshape(E, H), row, zeros64)
        h = _node_call(h, ag[0], ag[1],
                       wn1[:, :H].T, wn1[:, H:].T, _r(bn1, 2 * H),
                       wn2.T, _r(bn2, H))

    wg1, bg1 = params['gate1']
    wg2, bg2 = params['gate2']
    wg3, bg3 = params['gate3']
    wo, bo = params['output_proj']
    g3t = jnp.pad(wg3.T, ((0, 0), (0, 7)))
    g3b = jnp.pad(bg3.reshape(1, 1), ((0, 0), (0, 7)))
    return _head_call(h, bcol, wg1.T, _r(bg1, H),
                      _r(params['ln_g'], H), _r(params['ln_b'], H),
                      wg2.T, _r(bg2, H // 2), g3t, g3b,
                      wo.T, _r(bo, OUT))


# speedup vs baseline: 5.0413x; 1.0197x over previous
import functools

import jax
import jax.numpy as jnp
from jax import lax
from jax.experimental import pallas as pl
from jax.experimental.pallas import tpu as pltpu
from jax.experimental.pallas import tpu_sc as plsc

N = 10000
E = 320000
D_IN = 128
H = 64
OUT = 128
NPAD = 10240

NC = 2
NS = 16
NW = NC * NS
EPW = E // NW
K = 80
NIT = EPW // K
RPT = NPAD // NS

_mesh = plsc.VectorSubcoreMesh(core_axis_name="c", subcore_axis_name="s")
_sc_params = pltpu.CompilerParams(use_tc_tiling_on_sc=False)

f32 = jnp.float32
i32 = jnp.int32


@functools.partial(
    pl.kernel,
    out_type=(
        jax.ShapeDtypeStruct((E, 2 * H), f32),
        jax.ShapeDtypeStruct((E // 8, 128), f32),
    ),
    mesh=_mesh,
    compiler_params=_sc_params,
    scratch_types=[
        pltpu.VMEM((K,), i32),
        pltpu.VMEM((K,), i32),
        pltpu.VMEM((K,), i32),
        pltpu.VMEM((K,), i32),
        pltpu.VMEM((K, H), f32),
        pltpu.VMEM((K, H), f32),
        pltpu.VMEM((K, H), f32),
        pltpu.VMEM((K, H), f32),
        pltpu.VMEM((K, 16), f32),
        pltpu.VMEM((K, 16), f32),
        pltpu.VMEM((K, 16), f32),
        pltpu.VMEM((K, 16), f32),
        pltpu.VMEM((K // 8, 128), f32),
        pltpu.SemaphoreType.DMA,
        pltpu.SemaphoreType.DMA,
    ],
)
def _sc_gather4(h_hbm, c_hbm, row_hbm, col_hbm,
                xg_hbm, rel_hbm,
                ir0, ic0, ir1, ic1, bxr0, bxc0, bxr1, bxc1,
                bcr0, bcc0, bcr1, bcc1, brel, sem_g, sem_w):
    wid = lax.axis_index("s") * NC + lax.axis_index("c")
    base = wid * EPW
    L = 16

    def pack(bcr, bcc):
        def sub(r, carry2):
            brel[r // 8, pl.ds(L * (r % 8), L)] = bcr[r, :] - bcc[r, :]
            return carry2

        lax.fori_loop(0, K, sub, 0)

    def step(j, ir, ic, bxr, bxc, bcr, bcc,
             pir, pic, pxr, pxc, pcr, pcc):
        off = base + j * K
        g1 = pltpu.async_copy(h_hbm.at[ir], bxr, sem_g)
        g2 = pltpu.async_copy(h_hbm.at[ic], bxc, sem_g)
        g3 = pltpu.async_copy(c_hbm.at[ir], bcr, sem_g)
        g4 = pltpu.async_copy(c_hbm.at[ic], bcc, sem_g)

        @pl.when(j + 1 < NIT)
        def _():
            pltpu.sync_copy(row_hbm.at[pl.ds(off + K, K)], pir)
            pltpu.sync_copy(col_hbm.at[pl.ds(off + K, K)], pic)

        @pl.when(j > 0)
        def _():
            pack(pcr, pcc)
            pltpu.async_copy(pxr, xg_hbm.at[pl.ds(off - K, K),
                                            pl.ds(0, H)], sem_w)
            pltpu.async_copy(pxc, xg_hbm.at[pl.ds(off - K, K),
                                            pl.ds(H, H)], sem_w)
            pltpu.async_copy(brel, rel_hbm.at[pl.ds((off - K) // 8,
                                              K // 8)], sem_w)

        g1.wait(); g2.wait(); g3.wait(); g4.wait()

        @pl.when(j > 0)
        def _():
            dw1 = pltpu.make_async_copy(pxr, xg_hbm.at[pl.ds(off - K, K),
                                                       pl.ds(0, H)], sem_w)
            dw2 = pltpu.make_async_copy(pxc, xg_hbm.at[pl.ds(off - K, K),
                                                       pl.ds(H, H)], sem_w)
            dw3 = pltpu.make_async_copy(brel, rel_hbm.at[pl.ds((off - K) // 8,
                                                         K // 8)], sem_w)
            dw1.wait(); dw2.wait(); dw3.wait()

    pltpu.sync_copy(row_hbm.at[pl.ds(base, K)], ir0)
    pltpu.sync_copy(col_hbm.at[pl.ds(base, K)], ic0)

    def body(j, carry):
        @pl.when(j % 2 == 0)
        def _():
            step(j, ir0, ic0, bxr0, bxc0, bcr0, bcc0,
                 ir1, ic1, bxr1, bxc1, bcr1, bcc1)

        @pl.when(j % 2 == 1)
        def _():
            step(j, ir1, ic1, bxr1, bxc1, bcr1, bcc1,
                 ir0, ic0, bxr0, bxc0, bcr0, bcc0)

        return carry

    lax.fori_loop(0, NIT, body, 0)
    offl = base + (NIT - 1) * K
    pack(bcr0, bcc0)
    pltpu.sync_copy(bxr0, xg_hbm.at[pl.ds(offl, K), pl.ds(0, H)])
    pltpu.sync_copy(bxc0, xg_hbm.at[pl.ds(offl, K), pl.ds(H, H)])
    pltpu.sync_copy(brel, rel_hbm.at[pl.ds(offl // 8, K // 8)])


@functools.partial(
    pl.kernel,
    out_type=jax.ShapeDtypeStruct((E // 8, 128), f32),
    mesh=_mesh,
    compiler_params=_sc_params,
    scratch_types=[
        pltpu.VMEM((K,), i32),
        pltpu.VMEM((K,), i32),
        pltpu.VMEM((K,), i32),
        pltpu.VMEM((K,), i32),
        pltpu.VMEM((K, 16), f32),
        pltpu.VMEM((K, 16), f32),
        pltpu.VMEM((K, 16), f32),
        pltpu.VMEM((K, 16), f32),
        pltpu.VMEM((K // 8, 128), f32),
        pltpu.SemaphoreType.DMA,
        pltpu.SemaphoreType.DMA,
    ],
)
def _sc_gather2(c_hbm, row_hbm, col_hbm, rel_hbm,
                ir0, ic0, ir1, ic1, bcr0, bcc0, bcr1, bcc1,
                brel, sem_g, sem_w):
    wid = lax.axis_index("s") * NC + lax.axis_index("c")
    base = wid * EPW
    L = 16

    def pack(bcr, bcc):
        def sub(r, carry2):
            brel[r // 8, pl.ds(L * (r % 8), L)] = bcr[r, :] - bcc[r, :]
            return carry2

        lax.fori_loop(0, K, sub, 0)

    def step(j, ir, ic, bcr, bcc, pir, pic, pcr, pcc):
        off = base + j * K
        g1 = pltpu.async_copy(c_hbm.at[ir], bcr, sem_g)
        g2 = pltpu.async_copy(c_hbm.at[ic], bcc, sem_g)

        @pl.when(j + 1 < NIT)
        def _():
            pltpu.sync_copy(row_hbm.at[pl.ds(off + K, K)], pir)
            pltpu.sync_copy(col_hbm.at[pl.ds(off + K, K)], pic)

        @pl.when(j > 0)
        def _():
            pack(pcr, pcc)
            pltpu.async_copy(brel, rel_hbm.at[pl.ds((off - K) // 8,
                                              K // 8)], sem_w)

        g1.wait(); g2.wait()

        @pl.when(j > 0)
        def _():
            pltpu.make_async_copy(brel, rel_hbm.at[pl.ds((off - K) // 8,
                                                   K // 8)], sem_w).wait()

    pltpu.sync_copy(row_hbm.at[pl.ds(base, K)], ir0)
    pltpu.sync_copy(col_hbm.at[pl.ds(base, K)], ic0)

    def body(j, carry):
        @pl.when(j % 2 == 0)
        def _():
            step(j, ir0, ic0, bcr0, bcc0, ir1, ic1, bcr1, bcc1)

        @pl.when(j % 2 == 1)
        def _():
            step(j, ir1, ic1, bcr1, bcc1, ir0, ic0, bcr0, bcc0)

        return carry

    lax.fori_loop(0, NIT, body, 0)
    offl = base + (NIT - 1) * K
    pack(bcr0, bcc0)
    pltpu.sync_copy(brel, rel_hbm.at[pl.ds(offl // 8, K // 8)])


def _make_sc_scatter(W):
    @functools.partial(
        pl.kernel,
        out_type=jax.ShapeDtypeStruct((2, NPAD, W), f32),
        mesh=_mesh,
        compiler_params=_sc_params,
        scratch_types=[
            pltpu.VMEM((K,), i32),
            pltpu.VMEM((K,), i32),
            pltpu.VMEM((K, W), f32),
            pltpu.VMEM((K, W), f32),
            pltpu.VMEM_SHARED((NPAD, W), f32),
            pltpu.SemaphoreType.DMA,
            pltpu.SemaphoreType.DMA,
        ],
    )
    def scat(val_hbm, row_hbm, zero_hbm, out_hbm,
             i0, i1, v0, v1, acc, sem_s, sem_v):
        cid = lax.axis_index("c")
        sid = lax.axis_index("s")
        wid = sid * NC + cid
        pltpu.sync_copy(zero_hbm.at[pl.ds(sid * RPT, RPT)],
                        acc.at[pl.ds(sid * RPT, RPT)])
        plsc.subcore_barrier()
        base = wid * EPW

        def step(j, ci, cv, pi, pv):
            s = pltpu.async_copy(cv, acc.at[ci], sem_s, add=True)

            @pl.when(j + 1 < NIT)
            def _():
                off2 = base + (j + 1) * K
                l1 = pltpu.async_copy(row_hbm.at[pl.ds(off2, K)], pi, sem_v)
                l2 = pltpu.async_copy(val_hbm.at[pl.ds(off2, K)], pv, sem_v)
                l1.wait(); l2.wait()

            s.wait()

        pltpu.sync_copy(row_hbm.at[pl.ds(base, K)], i0)
        pltpu.sync_copy(val_hbm.at[pl.ds(base, K)], v0)

        def body(j, carry):
            @pl.when(j % 2 == 0)
            def _():
                step(j, i0, v0, i1, v1)

            @pl.when(j % 2 == 1)
            def _():
                step(j, i1, v1, i0, v0)

            return carry

        lax.fori_loop(0, NIT, body, 0)
        plsc.subcore_barrier()
        pltpu.sync_copy(acc.at[pl.ds(sid * RPT, RPT)],
                        out_hbm.at[cid, pl.ds(sid * RPT, RPT)])

    return scat


_sc_scatter16 = _make_sc_scatter(16)
_sc_scatter64 = _make_sc_scatter(H)


EB = 12800
NBK = 2048


def _full(shape):
    return pl.BlockSpec(shape, lambda *_: tuple(0 for _ in shape))


def _rows(bs, w):
    return pl.BlockSpec((bs, w), lambda i, *_: (i, 0))


def _proj_body(x_ref, wt, b, o_ref):
    o_ref[...] = (jnp.dot(x_ref[...], wt[...], preferred_element_type=f32)
                  + b[...])


def _edge_a_body(xg, relp, w1x, wxp, b1, w2b, mones, wz, msc, dout):
    urel = jnp.dot(relp[...], wxp[...],
                   preferred_element_type=f32).reshape(EB, 2 * H)
    u = jnp.dot(xg[...], w1x[...], preferred_element_type=f32) + urel + b1[...]
    u = u * jax.nn.sigmoid(u)
    up = u.reshape(EB // 8, 8 * 2 * H)
    dp = jnp.dot(up, w2b[...], preferred_element_type=f32)
    n2 = jnp.dot(dp * dp, mones[...], preferred_element_type=f32)
    nrm = jnp.maximum(jnp.sqrt(n2), 1e-8)
    ew = jax.nn.sigmoid(jnp.dot(relp[...], wz[...], preferred_element_type=f32)
                        + msc[0, 1])
    dout[...] = dp * ((msc[0, 0] * ew) / nrm)


def _edge_b_body(xg, relp, e1x, wxp, b1, e2b, b2p, eout):
    urel = jnp.dot(relp[...], wxp[...],
                   preferred_element_type=f32).reshape(EB, 2 * H)
    u = jnp.dot(xg[...], e1x[...], preferred_element_type=f32) + urel + b1[...]
    u = u * jax.nn.sigmoid(u)
    up = u.reshape(EB // 2, 2 * 2 * H)
    v = jnp.dot(up, e2b[...], preferred_element_type=f32) + b2p[...]
    eout[...] = v * jax.nn.sigmoid(v)


def _cupd_body(c, q0, q1, cout):
    cout[...] = c[...] + q0[...] + q1[...]


def _node_body(h, p0, p1, n1h, n1a, b1, n2t, b2, hout):
    agg = p0[...] + p1[...]
    u = (jnp.dot(h[...], n1h[...], preferred_element_type=f32)
         + jnp.dot(agg, n1a[...], preferred_element_type=f32) + b1[...])
    u = u * jax.nn.sigmoid(u)
    hout[...] = h[...] + jnp.dot(u, n2t[...], preferred_element_type=f32) + b2[...]


def _head_body(h, bcol, g1t, g1b, lng, lnb, g2t, g2b, g3t, g3b, wot, bo,
               out, m_ref, d_ref, p_ref):
    ph = pl.program_id(0)
    i = pl.program_id(1)
    G = pl.num_programs(1)

    @pl.when(jnp.logical_and(ph == 0, i == 0))
    def _():
        m_ref[...] = jnp.full((1, 8), -1e30, f32)

    @pl.when(jnp.logical_and(ph == 1, i == 0))
    def _():
        d_ref[...] = jnp.zeros((8, 8), f32)
        p_ref[...] = jnp.zeros((8, H), f32)

    g = jnp.dot(h[...], g1t[...], preferred_element_type=f32) + g1b[...]
    mu = jnp.mean(g, axis=-1, keepdims=True)
    var = jnp.mean((g - mu) ** 2, axis=-1, keepdims=True)
    g = (g - mu) * lax.rsqrt(var + 1e-5) * lng[...] + lnb[...]
    g = jnp.maximum(g, 0.0)
    q = jnp.maximum(jnp.dot(g, g2t[...], preferred_element_type=f32)
                    + g2b[...], 0.0)
    s = jnp.dot(q, g3t[...], preferred_element_type=f32) + g3b[...]
    s0 = s[:, 0:1]
    oh = bcol[...] == lax.broadcasted_iota(i32, (1, 8), 1)

    @pl.when(ph == 0)
    def _():
        sm = jnp.where(oh, s0, -1e30)
        bm = jnp.max(sm, axis=0, keepdims=True)
        m_ref[...] = jnp.maximum(m_ref[...], bm)

    @pl.when(ph == 1)
    def _():
        w = jnp.where(oh, jnp.exp(s0 - m_ref[...]), 0.0)
        dn = (((0,), (0,)), ((), ()))
        d_ref[...] += lax.dot_general(w, jnp.ones((NBK, 8), f32), dn,
                                      preferred_element_type=f32)
        p_ref[...] += lax.dot_general(w, h[...], dn,
                                      preferred_element_type=f32)

    @pl.when(jnp.logical_and(ph == 1, i == G - 1))
    def _():
        pooled = p_ref[...] / d_ref[:, 0:1]
        out[...] = (jnp.dot(pooled, wot[...], preferred_element_type=f32)
                    + bo[...])


def _proj_call(xp, wt, b):
    return pl.pallas_call(
        _proj_body,
        grid=(NPAD // NBK,),
        in_specs=[_rows(NBK, D_IN), _full((D_IN, H)), _full((1, H))],
        out_specs=_rows(NBK, H),
        out_shape=jax.ShapeDtypeStruct((NPAD, H), f32),
    )(xp, wt, b)


def _edge_a_call(xg, relp, w1x, wxp, b1, w2b, mones, wz, msc):
    return pl.pallas_call(
        _edge_a_body,
        grid=(E // EB,),
        in_specs=[_rows(EB, 2 * H), _rows(EB // 8, 128),
                  _full((2 * H, 2 * H)), _full((128, 8 * 2 * H)),
                  _full((1, 2 * H)), _full((8 * 2 * H, 128)),
                  _full((128, 128)), _full((128, 128)),
                  pl.BlockSpec(memory_space=pltpu.SMEM)],
        out_specs=_rows(EB // 8, 128),
        out_shape=jax.ShapeDtypeStruct((E // 8, 128), f32),
    )(xg, relp, w1x, wxp, b1, w2b, mones, wz, msc)


def _edge_b_call(xg, relp, e1x, wxp, b1, e2b, b2p):
    return pl.pallas_call(
        _edge_b_body,
        grid=(E // EB,),
        in_specs=[_rows(EB, 2 * H), _rows(EB // 8, 128),
                  _full((2 * H, 2 * H)), _full((128, 8 * 2 * H)),
                  _full((1, 2 * H)), _full((2 * 2 * H, 128)), _full((1, 128))],
        out_specs=_rows(EB // 2, 128),
        out_shape=jax.ShapeDtypeStruct((E // 2, 128), f32),
    )(xg, relp, e1x, wxp, b1, e2b, b2p)


def _cupd_call(c, q0, q1):
    return pl.pallas_call(
        _cupd_body,
        grid=(NPAD // NBK,),
        in_specs=[_rows(NBK, 16)] * 3,
        out_specs=_rows(NBK, 16),
        out_shape=jax.ShapeDtypeStruct((NPAD, 16), f32),
    )(c, q0, q1)


def _node_call(h, p0, p1, n1h, n1a, b1, n2t, b2):
    return pl.pallas_call(
        _node_body,
        grid=(NPAD // NBK,),
        in_specs=[_rows(NBK, H), _rows(NBK, H), _rows(NBK, H),
                  _full((H, 2 * H)), _full((H, 2 * H)), _full((1, 2 * H)),
                  _full((2 * H, H)), _full((1, H))],
        out_specs=_rows(NBK, H),
        out_shape=jax.ShapeDtypeStruct((NPAD, H), f32),
    )(h, p0, p1, n1h, n1a, b1, n2t, b2)


def _head_call(h, bcol, g1t, g1b, lng, lnb, g2t, g2b, g3t, g3b, wot, bo):
    return pl.pallas_call(
        _head_body,
        grid=(2, NPAD // NBK),
        in_specs=[pl.BlockSpec((NBK, H), lambda p, i: (i, 0)),
                  pl.BlockSpec((NBK, 1), lambda p, i: (i, 0)),
                  _full((H, H)), _full((1, H)), _full((1, H)), _full((1, H)),
                  _full((H, H // 2)), _full((1, H // 2)),
                  _full((H // 2, 8)), _full((1, 8)),
                  _full((H, OUT)), _full((1, OUT))],
        out_specs=pl.BlockSpec((8, OUT), lambda p, i: (0, 0)),
        out_shape=jax.ShapeDtypeStruct((8, OUT), f32),
        scratch_shapes=[pltpu.VMEM((1, 8), f32), pltpu.VMEM((8, 8), f32),
                        pltpu.VMEM((8, H), f32)],
    )(h, bcol, g1t, g1b, lng, lnb, g2t, g2b, g3t, g3b, wot, bo)


def _r(v, n):
    return v.reshape(1, n)


def _expand_rel(w3):
    wp = jnp.pad(w3, ((0, 13), (0, 0)))
    return jnp.einsum('qp,tc->qtpc', jnp.eye(8, dtype=f32),
                      wp).reshape(128, 8 * 128)


def _blockdiag(wt, k):
    a, b = wt.shape
    return jnp.einsum('pq,dc->pdqc', jnp.eye(k, dtype=f32),
                      wt).reshape(k * a, k * b)


def kernel(x, pos, edge_index, batch, params):
    row = edge_index[0].astype(i32)
    col = edge_index[1].astype(i32)

    xp = jnp.pad(x, ((0, NPAD - N), (0, 0)))
    coords = jnp.pad(pos, ((0, NPAD - N), (0, 13)))
    bcol = jnp.pad(batch.astype(i32), (0, NPAD - N),
                   constant_values=127).reshape(NPAD, 1)
    zeros16 = jnp.zeros((NPAD, 16), f32)
    zeros64 = jnp.zeros((NPAD, H), f32)

    wi, bi = params['input_proj']
    h = _proj_call(xp, wi.T, _r(bi, H))

    for p in params['layers']:
        w1, b1 = p['coord1']
        w2, b2 = p['coord2']
        we, be = p['ew']
        we1, be1 = p['edge1']
        we2, be2 = p['edge2']
        wn1, bn1 = p['node1']
        wn2, bn2 = p['node2']

        wxp_a = _expand_rel(w1[:, 2 * H:].T)
        wxp_b = _expand_rel(we1[:, 2 * H:].T)
        w2b = _blockdiag(jnp.pad(w2.T, ((0, 0), (0, 13))), 8)
        e2b = _blockdiag(we2.T, 2)
        b2p = jnp.tile(be2, 2).reshape(1, 128)
        ones16 = jnp.ones((16,), f32)
        mones = jnp.einsum('qp,t,c->qtpc', jnp.eye(8, dtype=f32),
                           ones16, ones16).reshape(128, 128)
        wz = jnp.einsum('qp,t,c->qtpc', jnp.eye(8, dtype=f32),
                        jnp.pad(we[0], (0, 13)), ones16).reshape(128, 128)
        msc = jnp.stack([p['scale'][0], be[0]]).reshape(1, 2)
        msc = jnp.pad(msc, ((0, 0), (0, 6)))

        xg, relp = _sc_gather4(h, coords, row, col)
        delta = _edge_a_call(xg, relp,
                             w1[:, :2 * H].T, wxp_a,
                             _r(b1, 2 * H), w2b, mones, wz, msc)
        q = _sc_scatter16(delta.reshape(E, 16), row, zeros16)
        coords = _cupd_call(coords, q[0], q[1])
        relp2 = _sc_gather2(coords, row, col)
        e = _edge_b_call(xg, relp2,
                         we1[:, :2 * H].T, wxp_b,
                         _r(be1, 2 * H), e2b, b2p)
        ag = _sc_scatter64(e.reshape(E, H), row, zeros64)
        h = _node_call(h, ag[0], ag[1],
                       wn1[:, :H].T, wn1[:, H:].T, _r(bn1, 2 * H),
                       wn2.T, _r(bn2, H))

    wg1, bg1 = params['gate1']
    wg2, bg2 = params['gate2']
    wg3, bg3 = params['gate3']
    wo, bo = params['output_proj']
    g3t = jnp.pad(wg3.T, ((0, 0), (0, 7)))
    g3b = jnp.pad(bg3.reshape(1, 1), ((0, 0), (0, 7)))
    return _head_call(h, bcol, wg1.T, _r(bg1, H),
                      _r(params['ln_g'], H), _r(params['ln_b'], H),
                      wg2.T, _r(bg2, H // 2), g3t, g3b,
                      wo.T, _r(bo, OUT))
